# baseline (speedup 1.0000x reference)
import functools

import jax
import jax.numpy as jnp
import numpy as np
from jax.experimental import pallas as pl
from jax.experimental.pallas import tpu as pltpu

N = 10000
E = 160000
NE = 10
C = 32
SH = 16
NB = 8
NG = 64
R_MAX = 5.0
AVG_NEIGH = 10.0
L_SLICES = [(0, 1), (1, 4), (4, 9), (9, 16)]
LOF = [0, 1, 1, 1, 2, 2, 2, 2, 2, 3, 3, 3, 3, 3, 3, 3]

EB = 2000
NBLK_E = E // EB
W = 256
NPAD = 11264
NBN = 1024
NBLK_N = NPAD // NBN

_INTERPRET = False


def _np_S():
    S = np.zeros((C * SH, C * 4), np.float32)
    for m in range(SH):
        l = LOF[m]
        for c in range(C):
            S[m * 32 + c, c * 4 + l] = 1.0
    return S


def _np_TILE():
    T = np.zeros((C, C * SH), np.float32)
    for m in range(SH):
        for c in range(C):
            T[c, m * 32 + c] = 1.0
    return T


def _np_PERM():
    P = np.zeros((C * SH, C * SH), np.float32)
    for m in range(SH):
        for c in range(C):
            P[m * 32 + c, c * 16 + m] = 1.0
    return P


_S_CONST = _np_S()
_TILE_CONST = _np_TILE()
_PERM_CONST = _np_PERM()
_W3PERM = np.array([[c * 4 + l for c in range(C)] for l in range(4)],
                   np.int32).reshape(-1)


def _sph_harm_cols(x, y, z):
    s3 = np.sqrt(3.0); s15 = np.sqrt(15.0); s5 = np.sqrt(5.0)
    c1 = np.sqrt(35.0 / 8.0); c2 = np.sqrt(105.0); c3 = np.sqrt(21.0 / 8.0)
    c4 = np.sqrt(7.0) / 2.0; c5 = np.sqrt(105.0) / 2.0
    one = jnp.ones_like(x)
    return [
        one,
        s3 * x, s3 * y, s3 * z,
        s15 * x * y, s15 * y * z, (s5 / 2.0) * (3.0 * z * z - one),
        s15 * x * z, (s15 / 2.0) * (x * x - y * y),
        c1 * y * (3.0 * x * x - y * y), c2 * x * y * z,
        c3 * y * (5.0 * z * z - one), c4 * z * (5.0 * z * z - 3.0 * one),
        c3 * x * (5.0 * z * z - one), c5 * z * (x * x - y * y),
        c1 * x * (x * x - 3.0 * y * y),
    ]


def _edge_kernel(bases_ref, ep_ref, ids_ref, sj_ref, W1_ref, W2_ref, W3p_ref,
                 out_ref, acc_ref):
    b = pl.program_id(0)

    @pl.when(b == 0)
    def _():
        acc_ref[...] = jnp.zeros_like(acc_ref)

    @pl.when(b >= NBLK_E)
    def _():
        off = pl.multiple_of((b - NBLK_E) * NBN, NBN)
        out_ref[...] = acc_ref[pl.ds(off, NBN), :]

    @pl.when(b < NBLK_E)
    def _():
        _edge_block(bases_ref, ep_ref, ids_ref, sj_ref, W1_ref, W2_ref,
                    W3p_ref, acc_ref, b)


def _edge_block(bases_ref, ep_ref, ids_ref, sj_ref, W1_ref, W2_ref, W3p_ref,
                acc_ref, b):
    ep = ep_ref[...]
    vec = ep[:, 0:3] - ep[:, 3:6]
    r2 = jnp.sum(vec * vec, axis=1, keepdims=True)
    r = jnp.sqrt(r2)
    rc = jnp.maximum(r, 1e-9)
    u = vec / rc
    ux = u[:, 0:1]; uy = u[:, 1:2]; uz = u[:, 2:3]
    Y = jnp.concatenate(_sph_harm_cols(ux, uy, uz), axis=1)

    n = jax.lax.broadcasted_iota(jnp.int32, (EB, NB), 1).astype(
        jnp.float32) + 1.0
    rb = np.float32(np.sqrt(2.0 / R_MAX)) * jnp.sin(
        n * (np.pi / R_MAX) * r) / rc
    p = 5.0
    ur = r * np.float32(1.0 / R_MAX)
    u5 = ur * ur * ur * ur * ur
    cut = (1.0 - 0.5 * (p + 1.0) * (p + 2.0) * u5 + p * (p + 2.0) * u5 * ur
           - 0.5 * p * (p + 1.0) * u5 * ur * ur)
    cut = cut * (ur < 1.0).astype(jnp.float32)
    ef = rb * cut

    h1 = jax.nn.silu(jax.lax.dot_general(
        ef, W1_ref[...], (((1,), (0,)), ((), ())),
        preferred_element_type=jnp.float32))
    h2 = jax.nn.silu(jax.lax.dot_general(
        h1, W2_ref[...], (((1,), (0,)), ((), ())),
        preferred_element_type=jnp.float32))
    R2 = jax.lax.dot_general(
        h2, W3p_ref[...], (((1,), (0,)), ((), ())),
        preferred_element_type=jnp.float32)

    sj = sj_ref[...]
    P0 = R2[:, 0:32] * sj
    P1 = R2[:, 32:64] * sj
    P2 = R2[:, 64:96] * sj
    P3 = R2[:, 96:128] * sj
    P = [P0, P1, P2, P3]
    cols = [P[LOF[m]] * Y[:, m:m + 1] for m in range(SH)]
    msg = jnp.concatenate(cols, axis=1).astype(jnp.bfloat16)

    ids = ids_ref[0]
    base = pl.multiple_of(bases_ref[b], 8)
    rel = jnp.broadcast_to(ids - base, (W, EB))
    iot = jax.lax.broadcasted_iota(jnp.int32, (W, EB), 0)
    onehotT = (rel == iot).astype(jnp.bfloat16)

    contrib = jax.lax.dot_general(
        onehotT, msg, (((1,), (0,)), ((), ())),
        preferred_element_type=jnp.float32)
    cur = acc_ref[pl.ds(base, W), :]
    acc_ref[pl.ds(base, W), :] = cur + contrib


def _edge_call(bases, ep, ids3, sj, W1, W2, W3p):
    cap = lambda b: jnp.minimum(b, NBLK_E - 1)
    spec = pltpu.PrefetchScalarGridSpec(
        num_scalar_prefetch=1,
        grid=(NBLK_E + NBLK_N,),
        in_specs=[
            pl.BlockSpec((EB, 8), lambda b, s: (cap(b), 0)),
            pl.BlockSpec((1, 1, EB), lambda b, s: (cap(b), 0, 0)),
            pl.BlockSpec((EB, C), lambda b, s: (cap(b), 0)),
            pl.BlockSpec((NB, 64), lambda b, s: (0, 0)),
            pl.BlockSpec((64, 64), lambda b, s: (0, 0)),
            pl.BlockSpec((64, 128), lambda b, s: (0, 0)),
        ],
        out_specs=pl.BlockSpec(
            (NBN, C * SH),
            lambda b, s: (jnp.maximum(b - NBLK_E, 0), 0)),
        scratch_shapes=[pltpu.VMEM((NPAD, C * SH), jnp.float32)],
    )
    return pl.pallas_call(
        _edge_kernel,
        grid_spec=spec,
        out_shape=jax.ShapeDtypeStruct((NPAD, C * SH), jnp.float32),
        compiler_params=pltpu.CompilerParams(
            dimension_semantics=("arbitrary",)),
        interpret=_INTERPRET,
    )(bases, ep, ids3, sj, W1, W2, W3p)


def _node_kernel_first(x_ref, agg_ref, Wemb_ref, BM_ref, S_ref, Wp_ref,
                       T_ref, BO_ref, BS_ref, ve_ref, nf_ref):
    _node_body(x_ref, agg_ref, None, Wemb_ref, BM_ref, S_ref, Wp_ref, T_ref,
               BO_ref, BS_ref, ve_ref, nf_ref, first=True)


def _node_kernel_rest(x_ref, agg_ref, nfin_ref, BM_ref, S_ref, Wp_ref,
                      T_ref, BO_ref, BS_ref, ve_ref, nf_ref):
    _node_body(x_ref, agg_ref, nfin_ref, None, BM_ref, S_ref, Wp_ref, T_ref,
               BO_ref, BS_ref, ve_ref, nf_ref, first=False)


def _node_body(x_ref, agg_ref, nfin_ref, Wemb_ref, BM_ref, S_ref, Wp_ref,
               T_ref, BO_ref, BS_ref, ve_ref, nf_ref, first):
    x = x_ref[...]
    agg = agg_ref[...]

    def mm(a, b):
        return jax.lax.dot_general(a, b, (((1,), (0,)), ((), ())),
                                   preferred_element_type=jnp.float32)

    if first:
        h = mm(x, Wemb_ref[...])
        nf_cur = jnp.concatenate(
            [h, jnp.zeros((NBN, C * SH - C), jnp.float32)], axis=1)
    else:
        nf_cur = nfin_ref[...]

    mid = mm(agg, BM_ref[...])
    inv = mm(mid * mid, S_ref[...])
    g = jax.nn.silu(mm(inv, Wp_ref[...]))
    gt = mm(g, T_ref[...])
    ew = mm(x, ve_ref[...])
    sc = mm(nf_cur, BS_ref[...]) * ew
    nf_ref[...] = mm(mid, BO_ref[...]) * gt + sc


def _node_call(x, agg, nfin, Wemb, BM, S, Wp, T, BO, BS, ve, first):
    full = lambda shape: pl.BlockSpec(shape, lambda b: (0, 0))
    in_specs = [
        pl.BlockSpec((NBN, 16), lambda b: (b, 0)),
        pl.BlockSpec((NBN, C * SH), lambda b: (b, 0)),
    ]
    args = [x, agg]
    if first:
        kern = _node_kernel_first
        in_specs.append(full((16, C)))
        args.append(Wemb)
    else:
        kern = _node_kernel_rest
        in_specs.append(pl.BlockSpec((NBN, C * SH), lambda b: (b, 0)))
        args.append(nfin)
    in_specs += [full((C * SH, C * SH)), full((C * SH, C * 4)),
                 full((C * 4, C)), full((C, C * SH)),
                 full((C * SH, C * SH)), full((C * SH, C * SH)),
                 full((16, 1))]
    args += [BM, S, Wp, T, BO, BS, ve]
    return pl.pallas_call(
        kern,
        grid=(NBLK_N,),
        in_specs=in_specs,
        out_specs=pl.BlockSpec((NBN, C * SH), lambda b: (b, 0)),
        out_shape=jax.ShapeDtypeStruct((NPAD, C * SH), jnp.float32),
        compiler_params=pltpu.CompilerParams(
            dimension_semantics=("arbitrary",)),
        interpret=_INTERPRET,
    )(*args)


def _pool_kernel(nf_ref, bat_ref, PERM_ref, out_ref, acc, cnt):
    b = pl.program_id(0)

    @pl.when(b == 0)
    def _():
        acc[...] = jnp.zeros_like(acc)
        cnt[...] = jnp.zeros_like(cnt)

    bat = bat_ref[0]
    iot = jax.lax.broadcasted_iota(jnp.int32, (NG, NBN), 0)
    oh = (jnp.broadcast_to(bat, (NG, NBN)) == iot).astype(jnp.bfloat16)
    nfb = nf_ref[...].astype(jnp.bfloat16)

    def mm(a, b_, pt=jnp.float32):
        return jax.lax.dot_general(a, b_, (((1,), (0,)), ((), ())),
                                   preferred_element_type=pt)

    acc[...] = acc[...] + mm(oh, nfb)
    ones = jnp.ones((NBN, 128), jnp.bfloat16)
    cnt[...] = cnt[...] + mm(oh, ones)

    @pl.when(b == NBLK_N - 1)
    def _():
        c = jnp.maximum(cnt[:, 0:1], 1.0)
        out_ref[...] = mm(acc[...] / c, PERM_ref[...])


def _pool_call(nf, bat3, PERM):
    return pl.pallas_call(
        _pool_kernel,
        grid=(NBLK_N,),
        in_specs=[
            pl.BlockSpec((NBN, C * SH), lambda b: (b, 0)),
            pl.BlockSpec((1, 1, NBN), lambda b: (b, 0, 0)),
            pl.BlockSpec((C * SH, C * SH), lambda b: (0, 0)),
        ],
        out_specs=pl.BlockSpec((NG, C * SH), lambda b: (0, 0)),
        out_shape=jax.ShapeDtypeStruct((NG, C * SH), jnp.float32),
        scratch_shapes=[pltpu.VMEM((NG, C * SH), jnp.float32),
                        pltpu.VMEM((NG, 128), jnp.float32)],
        compiler_params=pltpu.CompilerParams(
            dimension_semantics=("arbitrary",)),
        interpret=_INTERPRET,
    )(nf, bat3, PERM)


def _big_blockdiag(w4, scale=1.0):
    z = jnp.zeros((C, C), jnp.float32)
    return jnp.concatenate(
        [jnp.concatenate([z] * m + [w4[LOF[m]] * scale]
                         + [z] * (SH - 1 - m), axis=1)
         for m in range(SH)], axis=0)


def kernel(x, pos, batch, edge_index, W_embed, W1_0, W2_0, W3_0, mix_0,
           Wsc_0, velem_0, Wprod_0, Wout_0, W1_1, W2_1, W3_1, mix_1, Wsc_1,
           velem_1, Wprod_1, Wout_1):
    f32 = jnp.float32
    x = x.astype(f32)
    j = edge_index[0].astype(jnp.int32)
    i = edge_index[1].astype(jnp.int32)
    bat = batch.astype(jnp.int32)

    order = jnp.argsort(i)
    i_s = i[order]
    j_s = j[order]
    bases = ((i_s[::EB] // 8) * 8).astype(jnp.int32)
    ids3 = i_s.reshape(NBLK_E, 1, EB)

    ep = jnp.concatenate(
        [pos[j_s], pos[i_s], jnp.zeros((E, 2), f32)], axis=1)

    x_pad = jnp.pad(x, ((0, NPAD - N), (0, 16 - NE)))
    Wemb_pad = jnp.pad(W_embed.astype(f32), ((0, 16 - NE), (0, 0)))

    S = jnp.asarray(_S_CONST)
    T = jnp.asarray(_TILE_CONST)
    PERM = jnp.asarray(_PERM_CONST)

    h = x @ W_embed.astype(f32)
    bat3 = jnp.pad(bat, (0, NPAD - N), constant_values=NG).reshape(
        NBLK_N, 1, NBN)

    layer_params = [
        (W1_0, W2_0, W3_0, mix_0, Wsc_0, velem_0, Wprod_0, Wout_0),
        (W1_1, W2_1, W3_1, mix_1, Wsc_1, velem_1, Wprod_1, Wout_1),
    ]

    nf = None
    s_table = h
    for li, (W1, W2, W3, mix, Wsc, velem, Wprod, Wout) in enumerate(
            layer_params):
        W3p = W3.astype(f32)[:, _W3PERM]
        BM = _big_blockdiag(mix.astype(f32), scale=1.0 / AVG_NEIGH)
        BS = _big_blockdiag(Wsc.astype(f32))
        BO = _big_blockdiag(Wout.astype(f32))
        ve = jnp.pad(velem.astype(f32), (0, 16 - NE)).reshape(16, 1)

        sj = s_table[j_s]
        agg = _edge_call(bases, ep, ids3, sj, W1.astype(f32),
                         W2.astype(f32), W3p)
        nf = _node_call(x_pad, agg, nf, Wemb_pad, BM, S,
                        Wprod.astype(f32), T, BO, BS, ve, first=(li == 0))
        s_table = nf[:N, :C]

    return _pool_call(nf, bat3, PERM)


# speedup vs baseline: 1.7918x; 1.7918x over previous
import functools

import jax
import jax.numpy as jnp
import numpy as np
from jax.experimental import pallas as pl
from jax.experimental.pallas import tpu as pltpu

N = 10000
E = 160000
NE = 10
C = 32
SH = 16
NB = 8
NG = 64
R_MAX = 5.0
AVG_NEIGH = 10.0
L_SLICES = [(0, 1), (1, 4), (4, 9), (9, 16)]
LOF = [0, 1, 1, 1, 2, 2, 2, 2, 2, 3, 3, 3, 3, 3, 3, 3]

EB = 1280
NBLK_E = E // EB
W = 256
NPAD = 11264
NBN = 1024
NBLK_N = NPAD // NBN

_INTERPRET = False


def _np_S():
    S = np.zeros((C * SH, C * 4), np.float32)
    for m in range(SH):
        l = LOF[m]
        for c in range(C):
            S[m * 32 + c, c * 4 + l] = 1.0
    return S


def _np_TILE():
    T = np.zeros((C, C * SH), np.float32)
    for m in range(SH):
        for c in range(C):
            T[c, m * 32 + c] = 1.0
    return T


def _np_PERM():
    P = np.zeros((C * SH, C * SH), np.float32)
    for m in range(SH):
        for c in range(C):
            P[m * 32 + c, c * 16 + m] = 1.0
    return P


_S_CONST = _np_S()
_TILE_CONST = _np_TILE()
_PERM_CONST = _np_PERM()
_W3PERM = np.array([[c * 4 + l for c in range(C)] for l in range(4)],
                   np.int32).reshape(-1)


def _np_REP():
    M = np.zeros((C, 128), np.float32)
    for l in range(4):
        for c in range(C):
            M[c, l * 32 + c] = 1.0
    return M


def _np_SEL():
    M = np.zeros((128, C * SH), np.float32)
    for m in range(SH):
        for c in range(C):
            M[LOF[m] * 32 + c, m * 32 + c] = 1.0
    return M


def _np_EXPAND():
    M = np.zeros((SH, C * SH), np.float32)
    for m in range(SH):
        for c in range(C):
            M[m, m * 32 + c] = 1.0
    return M


_REP_CONST = _np_REP()
_SEL_CONST = _np_SEL().astype(np.float32)
_EXPAND_CONST = _np_EXPAND()


def _sph_harm_cols(x, y, z):
    s3 = np.sqrt(3.0); s15 = np.sqrt(15.0); s5 = np.sqrt(5.0)
    c1 = np.sqrt(35.0 / 8.0); c2 = np.sqrt(105.0); c3 = np.sqrt(21.0 / 8.0)
    c4 = np.sqrt(7.0) / 2.0; c5 = np.sqrt(105.0) / 2.0
    one = jnp.ones_like(x)
    return [
        one,
        s3 * x, s3 * y, s3 * z,
        s15 * x * y, s15 * y * z, (s5 / 2.0) * (3.0 * z * z - one),
        s15 * x * z, (s15 / 2.0) * (x * x - y * y),
        c1 * y * (3.0 * x * x - y * y), c2 * x * y * z,
        c3 * y * (5.0 * z * z - one), c4 * z * (5.0 * z * z - 3.0 * one),
        c3 * x * (5.0 * z * z - one), c5 * z * (x * x - y * y),
        c1 * x * (x * x - 3.0 * y * y),
    ]


def _edge_kernel(bases_ref, ept_ref, ids_ref, sj_ref, W1_ref, W2_ref,
                 W3p_ref, REP_ref, SEL_ref, EXP_ref, out_ref, acc_ref):
    b = pl.program_id(0)

    @pl.when(b == 0)
    def _():
        acc_ref[...] = jnp.zeros_like(acc_ref)

    @pl.when(b >= NBLK_E)
    def _():
        off = pl.multiple_of((b - NBLK_E) * NBN, NBN)
        out_ref[...] = acc_ref[pl.ds(off, NBN), :]

    @pl.when(b < NBLK_E)
    def _():
        _edge_block(bases_ref, ept_ref, ids_ref, sj_ref, W1_ref, W2_ref,
                    W3p_ref, REP_ref, SEL_ref, EXP_ref, acc_ref, b)


def _edge_block(bases_ref, ept_ref, ids_ref, sj_ref, W1_ref, W2_ref, W3p_ref,
                REP_ref, SEL_ref, EXP_ref, acc_ref, b):
    def mm(a, bm, pt=jnp.float32):
        return jax.lax.dot_general(a, bm, (((1,), (0,)), ((), ())),
                                   preferred_element_type=pt)

    ept = ept_ref[...]
    vx = ept[0:1, :] - ept[3:4, :]
    vy = ept[1:2, :] - ept[4:5, :]
    vz = ept[2:3, :] - ept[5:6, :]
    r2 = vx * vx + vy * vy + vz * vz
    r = jnp.sqrt(r2)
    rc = jnp.maximum(r, 1e-9)
    rinv = 1.0 / rc
    ux = vx * rinv; uy = vy * rinv; uz = vz * rinv
    YT = jnp.concatenate(_sph_harm_cols(ux, uy, uz), axis=0)

    p = 5.0
    ur = r * np.float32(1.0 / R_MAX)
    u5 = ur * ur * ur * ur * ur
    cut = (1.0 - 0.5 * (p + 1.0) * (p + 2.0) * u5 + p * (p + 2.0) * u5 * ur
           - 0.5 * p * (p + 1.0) * u5 * ur * ur)
    cut = cut * (ur < 1.0).astype(jnp.float32)
    scal = np.float32(np.sqrt(2.0 / R_MAX))
    amp = scal * rinv * cut
    efT = jnp.concatenate(
        [jnp.sin(np.float32(n * np.pi / R_MAX) * r) * amp
         for n in range(1, NB + 1)], axis=0)

    ef = jnp.transpose(efT)
    Y = jnp.transpose(YT)

    h1 = jax.nn.silu(mm(ef, W1_ref[...]))
    h2 = jax.nn.silu(mm(h1, W2_ref[...]))
    R2 = mm(h2, W3p_ref[...])

    sj = sj_ref[...]
    T = R2 * mm(sj, REP_ref[...])
    Tsel = mm(T.astype(jnp.bfloat16), SEL_ref[...]).astype(jnp.bfloat16)
    Yexp = mm(Y.astype(jnp.bfloat16), EXP_ref[...]).astype(jnp.bfloat16)
    msg = Tsel * Yexp

    ids = ids_ref[0]
    base = pl.multiple_of(bases_ref[b], 8)
    rel = jnp.broadcast_to(ids - base, (W, EB))
    iot = jax.lax.broadcasted_iota(jnp.int32, (W, EB), 0)
    onehotT = (rel == iot).astype(jnp.bfloat16)

    contrib = jax.lax.dot_general(
        onehotT, msg, (((1,), (0,)), ((), ())),
        preferred_element_type=jnp.float32)
    cur = acc_ref[pl.ds(base, W), :]
    acc_ref[pl.ds(base, W), :] = cur + contrib


def _edge_call(bases, ept, ids3, sj, W1, W2, W3p, REPc, SELc, EXPc):
    cap = lambda b: jnp.minimum(b, NBLK_E - 1)
    spec = pltpu.PrefetchScalarGridSpec(
        num_scalar_prefetch=1,
        grid=(NBLK_E + NBLK_N,),
        in_specs=[
            pl.BlockSpec((8, EB), lambda b, s: (0, cap(b))),
            pl.BlockSpec((1, 1, EB), lambda b, s: (cap(b), 0, 0)),
            pl.BlockSpec((EB, C), lambda b, s: (cap(b), 0)),
            pl.BlockSpec((NB, 64), lambda b, s: (0, 0)),
            pl.BlockSpec((64, 64), lambda b, s: (0, 0)),
            pl.BlockSpec((64, 128), lambda b, s: (0, 0)),
            pl.BlockSpec((C, 128), lambda b, s: (0, 0)),
            pl.BlockSpec((128, C * SH), lambda b, s: (0, 0)),
            pl.BlockSpec((SH, C * SH), lambda b, s: (0, 0)),
        ],
        out_specs=pl.BlockSpec(
            (NBN, C * SH),
            lambda b, s: (jnp.maximum(b - NBLK_E, 0), 0)),
        scratch_shapes=[pltpu.VMEM((NPAD, C * SH), jnp.float32)],
    )
    return pl.pallas_call(
        _edge_kernel,
        grid_spec=spec,
        out_shape=jax.ShapeDtypeStruct((NPAD, C * SH), jnp.float32),
        compiler_params=pltpu.CompilerParams(
            dimension_semantics=("arbitrary",)),
        interpret=_INTERPRET,
    )(bases, ept, ids3, sj, W1, W2, W3p, REPc, SELc, EXPc)


def _node_kernel_first(x_ref, agg_ref, Wemb_ref, BM_ref, S_ref, Wp_ref,
                       T_ref, BO_ref, BS_ref, ve_ref, nf_ref):
    _node_body(x_ref, agg_ref, None, Wemb_ref, BM_ref, S_ref, Wp_ref, T_ref,
               BO_ref, BS_ref, ve_ref, nf_ref, first=True)


def _node_kernel_rest(x_ref, agg_ref, nfin_ref, BM_ref, S_ref, Wp_ref,
                      T_ref, BO_ref, BS_ref, ve_ref, nf_ref):
    _node_body(x_ref, agg_ref, nfin_ref, None, BM_ref, S_ref, Wp_ref, T_ref,
               BO_ref, BS_ref, ve_ref, nf_ref, first=False)


def _node_body(x_ref, agg_ref, nfin_ref, Wemb_ref, BM_ref, S_ref, Wp_ref,
               T_ref, BO_ref, BS_ref, ve_ref, nf_ref, first):
    x = x_ref[...]
    agg = agg_ref[...]

    def mm(a, b):
        return jax.lax.dot_general(a, b, (((1,), (0,)), ((), ())),
                                   preferred_element_type=jnp.float32)

    if first:
        h = mm(x, Wemb_ref[...])
        nf_cur = jnp.concatenate(
            [h, jnp.zeros((NBN, C * SH - C), jnp.float32)], axis=1)
    else:
        nf_cur = nfin_ref[...]

    mid = mm(agg, BM_ref[...])
    inv = mm(mid * mid, S_ref[...])
    g = jax.nn.silu(mm(inv, Wp_ref[...]))
    gt = mm(g, T_ref[...])
    ew = mm(x, ve_ref[...])
    sc = mm(nf_cur, BS_ref[...]) * ew
    nf_ref[...] = mm(mid, BO_ref[...]) * gt + sc


def _node_call(x, agg, nfin, Wemb, BM, S, Wp, T, BO, BS, ve, first):
    full = lambda shape: pl.BlockSpec(shape, lambda b: (0, 0))
    in_specs = [
        pl.BlockSpec((NBN, 16), lambda b: (b, 0)),
        pl.BlockSpec((NBN, C * SH), lambda b: (b, 0)),
    ]
    args = [x, agg]
    if first:
        kern = _node_kernel_first
        in_specs.append(full((16, C)))
        args.append(Wemb)
    else:
        kern = _node_kernel_rest
        in_specs.append(pl.BlockSpec((NBN, C * SH), lambda b: (b, 0)))
        args.append(nfin)
    in_specs += [full((C * SH, C * SH)), full((C * SH, C * 4)),
                 full((C * 4, C)), full((C, C * SH)),
                 full((C * SH, C * SH)), full((C * SH, C * SH)),
                 full((16, 1))]
    args += [BM, S, Wp, T, BO, BS, ve]
    return pl.pallas_call(
        kern,
        grid=(NBLK_N,),
        in_specs=in_specs,
        out_specs=pl.BlockSpec((NBN, C * SH), lambda b: (b, 0)),
        out_shape=jax.ShapeDtypeStruct((NPAD, C * SH), jnp.float32),
        compiler_params=pltpu.CompilerParams(
            dimension_semantics=("arbitrary",)),
        interpret=_INTERPRET,
    )(*args)


def _pool_kernel(nf_ref, bat_ref, PERM_ref, out_ref, acc, cnt):
    b = pl.program_id(0)

    @pl.when(b == 0)
    def _():
        acc[...] = jnp.zeros_like(acc)
        cnt[...] = jnp.zeros_like(cnt)

    bat = bat_ref[0]
    iot = jax.lax.broadcasted_iota(jnp.int32, (NG, NBN), 0)
    oh = (jnp.broadcast_to(bat, (NG, NBN)) == iot).astype(jnp.bfloat16)
    nfb = nf_ref[...].astype(jnp.bfloat16)

    def mm(a, b_, pt=jnp.float32):
        return jax.lax.dot_general(a, b_, (((1,), (0,)), ((), ())),
                                   preferred_element_type=pt)

    acc[...] = acc[...] + mm(oh, nfb)
    ones = jnp.ones((NBN, 128), jnp.bfloat16)
    cnt[...] = cnt[...] + mm(oh, ones)

    @pl.when(b == NBLK_N - 1)
    def _():
        c = jnp.maximum(cnt[:, 0:1], 1.0)
        out_ref[...] = mm(acc[...] / c, PERM_ref[...])


def _pool_call(nf, bat3, PERM):
    return pl.pallas_call(
        _pool_kernel,
        grid=(NBLK_N,),
        in_specs=[
            pl.BlockSpec((NBN, C * SH), lambda b: (b, 0)),
            pl.BlockSpec((1, 1, NBN), lambda b: (b, 0, 0)),
            pl.BlockSpec((C * SH, C * SH), lambda b: (0, 0)),
        ],
        out_specs=pl.BlockSpec((NG, C * SH), lambda b: (0, 0)),
        out_shape=jax.ShapeDtypeStruct((NG, C * SH), jnp.float32),
        scratch_shapes=[pltpu.VMEM((NG, C * SH), jnp.float32),
                        pltpu.VMEM((NG, 128), jnp.float32)],
        compiler_params=pltpu.CompilerParams(
            dimension_semantics=("arbitrary",)),
        interpret=_INTERPRET,
    )(nf, bat3, PERM)


def _big_blockdiag(w4, scale=1.0):
    z = jnp.zeros((C, C), jnp.float32)
    return jnp.concatenate(
        [jnp.concatenate([z] * m + [w4[LOF[m]] * scale]
                         + [z] * (SH - 1 - m), axis=1)
         for m in range(SH)], axis=0)


def kernel(x, pos, batch, edge_index, W_embed, W1_0, W2_0, W3_0, mix_0,
           Wsc_0, velem_0, Wprod_0, Wout_0, W1_1, W2_1, W3_1, mix_1, Wsc_1,
           velem_1, Wprod_1, Wout_1):
    f32 = jnp.float32
    x = x.astype(f32)
    j = edge_index[0].astype(jnp.int32)
    i = edge_index[1].astype(jnp.int32)
    bat = batch.astype(jnp.int32)

    order = jnp.argsort(i)
    i_s = i[order]
    j_s = j[order]
    bases = ((i_s[::EB] // 8) * 8).astype(jnp.int32)
    ids3 = i_s.reshape(NBLK_E, 1, EB)

    ept = jnp.concatenate(
        [pos[j_s].T, pos[i_s].T, jnp.zeros((2, E), f32)], axis=0)
    REPc = jnp.asarray(_REP_CONST)
    SELc = jnp.asarray(_SEL_CONST).astype(jnp.bfloat16)
    EXPc = jnp.asarray(_EXPAND_CONST).astype(jnp.bfloat16)

    x_pad = jnp.pad(x, ((0, NPAD - N), (0, 16 - NE)))
    Wemb_pad = jnp.pad(W_embed.astype(f32), ((0, 16 - NE), (0, 0)))

    S = jnp.asarray(_S_CONST)
    T = jnp.asarray(_TILE_CONST)
    PERM = jnp.asarray(_PERM_CONST)

    h = x @ W_embed.astype(f32)
    bat3 = jnp.pad(bat, (0, NPAD - N), constant_values=NG).reshape(
        NBLK_N, 1, NBN)

    layer_params = [
        (W1_0, W2_0, W3_0, mix_0, Wsc_0, velem_0, Wprod_0, Wout_0),
        (W1_1, W2_1, W3_1, mix_1, Wsc_1, velem_1, Wprod_1, Wout_1),
    ]

    nf = None
    s_table = h
    for li, (W1, W2, W3, mix, Wsc, velem, Wprod, Wout) in enumerate(
            layer_params):
        W3p = W3.astype(f32)[:, _W3PERM]
        BM = _big_blockdiag(mix.astype(f32), scale=1.0 / AVG_NEIGH)
        BS = _big_blockdiag(Wsc.astype(f32))
        BO = _big_blockdiag(Wout.astype(f32))
        ve = jnp.pad(velem.astype(f32), (0, 16 - NE)).reshape(16, 1)

        sj = s_table[j_s]
        agg = _edge_call(bases, ept, ids3, sj, W1.astype(f32),
                         W2.astype(f32), W3p, REPc, SELc, EXPc)
        nf = _node_call(x_pad, agg, nf, Wemb_pad, BM, S,
                        Wprod.astype(f32), T, BO, BS, ve, first=(li == 0))
        s_table = nf[:N, :C]

    return _pool_call(nf, bat3, PERM)


# speedup vs baseline: 2.8406x; 1.5854x over previous
import functools

import jax
import jax.numpy as jnp
import numpy as np
from jax import lax
from jax.experimental import pallas as pl
from jax.experimental.pallas import tpu as pltpu
from jax.experimental.pallas import tpu_sc as plsc

N = 10000
E = 160000
NE = 10
C = 32
SH = 16
NB = 8
NG = 64
R_MAX = 5.0
AVG_NEIGH = 10.0
L_SLICES = [(0, 1), (1, 4), (4, 9), (9, 16)]
LOF = [0, 1, 1, 1, 2, 2, 2, 2, 2, 3, 3, 3, 3, 3, 3, 3]

EB = 1280
NBLK_E = E // EB
W = 256
NPAD = 11264
NBN = 1024
NBLK_N = NPAD // NBN

_INTERPRET = False


def _np_S():
    S = np.zeros((C * SH, C * 4), np.float32)
    for m in range(SH):
        l = LOF[m]
        for c in range(C):
            S[m * 32 + c, c * 4 + l] = 1.0
    return S


def _np_TILE():
    T = np.zeros((C, C * SH), np.float32)
    for m in range(SH):
        for c in range(C):
            T[c, m * 32 + c] = 1.0
    return T


def _np_PERM():
    P = np.zeros((C * SH, C * SH), np.float32)
    for m in range(SH):
        for c in range(C):
            P[m * 32 + c, c * 16 + m] = 1.0
    return P


_S_CONST = _np_S()
_TILE_CONST = _np_TILE()
_PERM_CONST = _np_PERM()
_W3PERM = np.array([[c * 4 + l for c in range(C)] for l in range(4)],
                   np.int32).reshape(-1)


def _np_REP():
    M = np.zeros((C, 128), np.float32)
    for l in range(4):
        for c in range(C):
            M[c, l * 32 + c] = 1.0
    return M


def _np_SEL():
    M = np.zeros((128, C * SH), np.float32)
    for m in range(SH):
        for c in range(C):
            M[LOF[m] * 32 + c, m * 32 + c] = 1.0
    return M


def _np_EXPAND():
    M = np.zeros((SH, C * SH), np.float32)
    for m in range(SH):
        for c in range(C):
            M[m, m * 32 + c] = 1.0
    return M


_REP_CONST = _np_REP()
_SEL_CONST = _np_SEL().astype(np.float32)
_EXPAND_CONST = _np_EXPAND()


def _sph_harm_cols(x, y, z):
    s3 = np.sqrt(3.0); s15 = np.sqrt(15.0); s5 = np.sqrt(5.0)
    c1 = np.sqrt(35.0 / 8.0); c2 = np.sqrt(105.0); c3 = np.sqrt(21.0 / 8.0)
    c4 = np.sqrt(7.0) / 2.0; c5 = np.sqrt(105.0) / 2.0
    one = jnp.ones_like(x)
    return [
        one,
        s3 * x, s3 * y, s3 * z,
        s15 * x * y, s15 * y * z, (s5 / 2.0) * (3.0 * z * z - one),
        s15 * x * z, (s15 / 2.0) * (x * x - y * y),
        c1 * y * (3.0 * x * x - y * y), c2 * x * y * z,
        c3 * y * (5.0 * z * z - one), c4 * z * (5.0 * z * z - 3.0 * one),
        c3 * x * (5.0 * z * z - one), c5 * z * (x * x - y * y),
        c1 * x * (x * x - 3.0 * y * y),
    ]


def _edge_kernel(bases_ref, pj_ref, pi_ref, ids_ref, sj_ref, W1_ref, W2_ref,
                 W3p_ref, REP_ref, SEL_ref, EXP_ref, out_ref, acc_ref):
    b = pl.program_id(0)

    @pl.when(b == 0)
    def _():
        acc_ref[...] = jnp.zeros_like(acc_ref)

    @pl.when(b >= NBLK_E)
    def _():
        off = pl.multiple_of((b - NBLK_E) * NBN, NBN)
        out_ref[...] = acc_ref[pl.ds(off, NBN), :]

    @pl.when(b < NBLK_E)
    def _():
        _edge_block(bases_ref, pj_ref, pi_ref, ids_ref, sj_ref, W1_ref,
                    W2_ref, W3p_ref, REP_ref, SEL_ref, EXP_ref, acc_ref, b)


def _edge_block(bases_ref, pj_ref, pi_ref, ids_ref, sj_ref, W1_ref, W2_ref,
                W3p_ref, REP_ref, SEL_ref, EXP_ref, acc_ref, b):
    def mm(a, bm, pt=jnp.float32):
        return jax.lax.dot_general(a, bm, (((1,), (0,)), ((), ())),
                                   preferred_element_type=pt)

    dT = jnp.transpose(pj_ref[:, :16] - pi_ref[:, :16])
    vx = dT[0:1, :]
    vy = dT[1:2, :]
    vz = dT[2:3, :]
    r2 = vx * vx + vy * vy + vz * vz
    r = jnp.sqrt(r2)
    rc = jnp.maximum(r, 1e-9)
    rinv = 1.0 / rc
    ux = vx * rinv; uy = vy * rinv; uz = vz * rinv
    YT = jnp.concatenate(_sph_harm_cols(ux, uy, uz), axis=0)

    p = 5.0
    ur = r * np.float32(1.0 / R_MAX)
    u5 = ur * ur * ur * ur * ur
    cut = (1.0 - 0.5 * (p + 1.0) * (p + 2.0) * u5 + p * (p + 2.0) * u5 * ur
           - 0.5 * p * (p + 1.0) * u5 * ur * ur)
    cut = cut * (ur < 1.0).astype(jnp.float32)
    scal = np.float32(np.sqrt(2.0 / R_MAX))
    amp = scal * rinv * cut
    efT = jnp.concatenate(
        [jnp.sin(np.float32(n * np.pi / R_MAX) * r) * amp
         for n in range(1, NB + 1)], axis=0)

    ef = jnp.transpose(efT)
    Y = jnp.transpose(YT)

    h1 = jax.nn.silu(mm(ef, W1_ref[...]))
    h2 = jax.nn.silu(mm(h1, W2_ref[...]))
    R2 = mm(h2, W3p_ref[...])

    sj = sj_ref[:, :C]
    T = R2 * mm(sj, REP_ref[...])
    Tsel = mm(T.astype(jnp.bfloat16), SEL_ref[...]).astype(jnp.bfloat16)
    Yexp = mm(Y.astype(jnp.bfloat16), EXP_ref[...]).astype(jnp.bfloat16)
    msg = Tsel * Yexp

    ids = ids_ref[0]
    base = pl.multiple_of(bases_ref[b], 8)
    rel = jnp.broadcast_to(ids - base, (W, EB))
    iot = jax.lax.broadcasted_iota(jnp.int32, (W, EB), 0)
    onehotT = (rel == iot).astype(jnp.bfloat16)

    contrib = jax.lax.dot_general(
        onehotT, msg, (((1,), (0,)), ((), ())),
        preferred_element_type=jnp.float32)
    cur = acc_ref[pl.ds(base, W), :]
    acc_ref[pl.ds(base, W), :] = cur + contrib


def _edge_call(bases, ppj, ppi, ids3, sj, W1, W2, W3p, REPc, SELc, EXPc):
    cap = lambda b: jnp.minimum(b, NBLK_E - 1)
    spec = pltpu.PrefetchScalarGridSpec(
        num_scalar_prefetch=1,
        grid=(NBLK_E + NBLK_N,),
        in_specs=[
            pl.BlockSpec((EB, 128), lambda b, s: (cap(b), 0)),
            pl.BlockSpec((EB, 128), lambda b, s: (cap(b), 0)),
            pl.BlockSpec((1, 1, EB), lambda b, s: (cap(b), 0, 0)),
            pl.BlockSpec((EB, 128), lambda b, s: (cap(b), 0)),
            pl.BlockSpec((NB, 64), lambda b, s: (0, 0)),
            pl.BlockSpec((64, 64), lambda b, s: (0, 0)),
            pl.BlockSpec((64, 128), lambda b, s: (0, 0)),
            pl.BlockSpec((C, 128), lambda b, s: (0, 0)),
            pl.BlockSpec((128, C * SH), lambda b, s: (0, 0)),
            pl.BlockSpec((SH, C * SH), lambda b, s: (0, 0)),
        ],
        out_specs=pl.BlockSpec(
            (NBN, C * SH),
            lambda b, s: (jnp.maximum(b - NBLK_E, 0), 0)),
        scratch_shapes=[pltpu.VMEM((NPAD, C * SH), jnp.float32)],
    )
    return pl.pallas_call(
        _edge_kernel,
        grid_spec=spec,
        out_shape=jax.ShapeDtypeStruct((NPAD, C * SH), jnp.float32),
        compiler_params=pltpu.CompilerParams(
            dimension_semantics=("arbitrary",)),
        interpret=_INTERPRET,
    )(bases, ppj, ppi, ids3, sj, W1, W2, W3p, REPc, SELc, EXPc)


def _node_kernel_first(x_ref, agg_ref, Wemb_ref, BM_ref, S_ref, Wp_ref,
                       T_ref, BO_ref, BS_ref, ve_ref, nf_ref):
    _node_body(x_ref, agg_ref, None, Wemb_ref, BM_ref, S_ref, Wp_ref, T_ref,
               BO_ref, BS_ref, ve_ref, nf_ref, first=True)


def _node_kernel_rest(x_ref, agg_ref, nfin_ref, BM_ref, S_ref, Wp_ref,
                      T_ref, BO_ref, BS_ref, ve_ref, nf_ref):
    _node_body(x_ref, agg_ref, nfin_ref, None, BM_ref, S_ref, Wp_ref, T_ref,
               BO_ref, BS_ref, ve_ref, nf_ref, first=False)


def _node_body(x_ref, agg_ref, nfin_ref, Wemb_ref, BM_ref, S_ref, Wp_ref,
               T_ref, BO_ref, BS_ref, ve_ref, nf_ref, first):
    x = x_ref[...]
    agg = agg_ref[...]

    def mm(a, b):
        return jax.lax.dot_general(a, b, (((1,), (0,)), ((), ())),
                                   preferred_element_type=jnp.float32)

    if first:
        h = mm(x, Wemb_ref[...])
        nf_cur = jnp.concatenate(
            [h, jnp.zeros((NBN, C * SH - C), jnp.float32)], axis=1)
    else:
        nf_cur = nfin_ref[...]

    mid = mm(agg, BM_ref[...])
    inv = mm(mid * mid, S_ref[...])
    g = jax.nn.silu(mm(inv, Wp_ref[...]))
    gt = mm(g, T_ref[...])
    ew = mm(x, ve_ref[...])
    sc = mm(nf_cur, BS_ref[...]) * ew
    nf_ref[...] = mm(mid, BO_ref[...]) * gt + sc


def _node_call(x, agg, nfin, Wemb, BM, S, Wp, T, BO, BS, ve, first):
    full = lambda shape: pl.BlockSpec(shape, lambda b: (0, 0))
    in_specs = [
        pl.BlockSpec((NBN, 16), lambda b: (b, 0)),
        pl.BlockSpec((NBN, C * SH), lambda b: (b, 0)),
    ]
    args = [x, agg]
    if first:
        kern = _node_kernel_first
        in_specs.append(full((16, C)))
        args.append(Wemb)
    else:
        kern = _node_kernel_rest
        in_specs.append(pl.BlockSpec((NBN, C * SH), lambda b: (b, 0)))
        args.append(nfin)
    in_specs += [full((C * SH, C * SH)), full((C * SH, C * 4)),
                 full((C * 4, C)), full((C, C * SH)),
                 full((C * SH, C * SH)), full((C * SH, C * SH)),
                 full((16, 1))]
    args += [BM, S, Wp, T, BO, BS, ve]
    return pl.pallas_call(
        kern,
        grid=(NBLK_N,),
        in_specs=in_specs,
        out_specs=pl.BlockSpec((NBN, C * SH), lambda b: (b, 0)),
        out_shape=jax.ShapeDtypeStruct((NPAD, C * SH), jnp.float32),
        compiler_params=pltpu.CompilerParams(
            dimension_semantics=("arbitrary",)),
        interpret=_INTERPRET,
    )(*args)


def _pool_kernel(nf_ref, bat_ref, PERM_ref, out_ref, acc, cnt):
    b = pl.program_id(0)

    @pl.when(b == 0)
    def _():
        acc[...] = jnp.zeros_like(acc)
        cnt[...] = jnp.zeros_like(cnt)

    bat = bat_ref[0]
    iot = jax.lax.broadcasted_iota(jnp.int32, (NG, NBN), 0)
    oh = (jnp.broadcast_to(bat, (NG, NBN)) == iot).astype(jnp.bfloat16)
    nfb = nf_ref[...].astype(jnp.bfloat16)

    def mm(a, b_, pt=jnp.float32):
        return jax.lax.dot_general(a, b_, (((1,), (0,)), ((), ())),
                                   preferred_element_type=pt)

    acc[...] = acc[...] + mm(oh, nfb)
    ones = jnp.ones((NBN, 128), jnp.bfloat16)
    cnt[...] = cnt[...] + mm(oh, ones)

    @pl.when(b == NBLK_N - 1)
    def _():
        c = jnp.maximum(cnt[:, 0:1], 1.0)
        out_ref[...] = mm(acc[...] / c, PERM_ref[...])


def _pool_call(nf, bat3, PERM):
    return pl.pallas_call(
        _pool_kernel,
        grid=(NBLK_N,),
        in_specs=[
            pl.BlockSpec((NBN, C * SH), lambda b: (b, 0)),
            pl.BlockSpec((1, 1, NBN), lambda b: (b, 0, 0)),
            pl.BlockSpec((C * SH, C * SH), lambda b: (0, 0)),
        ],
        out_specs=pl.BlockSpec((NG, C * SH), lambda b: (0, 0)),
        out_shape=jax.ShapeDtypeStruct((NG, C * SH), jnp.float32),
        scratch_shapes=[pltpu.VMEM((NG, C * SH), jnp.float32),
                        pltpu.VMEM((NG, 128), jnp.float32)],
        compiler_params=pltpu.CompilerParams(
            dimension_semantics=("arbitrary",)),
        interpret=_INTERPRET,
    )(nf, bat3, PERM)


def _sc_gather(table, idx, chunk):
    V, D = table.shape
    B = idx.shape[0]
    NW = 32
    b_per_w = B // NW
    nchunk = b_per_w // chunk
    mesh = plsc.VectorSubcoreMesh(core_axis_name="c", subcore_axis_name="s")

    @functools.partial(
        pl.kernel, mesh=mesh,
        out_type=jax.ShapeDtypeStruct((B, D), jnp.float32),
        compiler_params=pltpu.CompilerParams(use_tc_tiling_on_sc=True),
        scratch_types=[
            pltpu.VMEM((chunk,), jnp.int32),
            pltpu.VMEM((chunk, D), jnp.float32),
            pltpu.SemaphoreType.DMA,
        ],
    )
    def k(table_hbm, idx_hbm, out_hbm, idx_v, rows_v, sem):
        wid = lax.axis_index("s") * 2 + lax.axis_index("c")
        for ci in range(nchunk):
            base = wid * b_per_w + ci * chunk
            pltpu.sync_copy(idx_hbm.at[pl.ds(base, chunk)], idx_v)
            pltpu.async_copy(table_hbm.at[idx_v], rows_v, sem).wait()
            pltpu.sync_copy(rows_v, out_hbm.at[pl.ds(base, chunk)])

    return k(table, idx)


def _embed_kernel(x_ref, W_ref, h_ref):
    h_ref[...] = jax.lax.dot_general(
        x_ref[...], W_ref[...], (((1,), (0,)), ((), ())),
        preferred_element_type=jnp.float32)


def _embed_call(x_pad, Wemb128):
    return pl.pallas_call(
        _embed_kernel,
        out_shape=jax.ShapeDtypeStruct((NPAD, 128), jnp.float32),
        interpret=_INTERPRET,
    )(x_pad, Wemb128)


def _big_blockdiag(w4, scale=1.0):
    z = jnp.zeros((C, C), jnp.float32)
    return jnp.concatenate(
        [jnp.concatenate([z] * m + [w4[LOF[m]] * scale]
                         + [z] * (SH - 1 - m), axis=1)
         for m in range(SH)], axis=0)


def kernel(x, pos, batch, edge_index, W_embed, W1_0, W2_0, W3_0, mix_0,
           Wsc_0, velem_0, Wprod_0, Wout_0, W1_1, W2_1, W3_1, mix_1, Wsc_1,
           velem_1, Wprod_1, Wout_1):
    f32 = jnp.float32
    x = x.astype(f32)
    j = edge_index[0].astype(jnp.int32)
    i = edge_index[1].astype(jnp.int32)
    bat = batch.astype(jnp.int32)

    order = jnp.argsort(i)
    i_s = i[order]
    j_s = j[order]
    bases = ((i_s[::EB] // 8) * 8).astype(jnp.int32)
    ids3 = i_s.reshape(NBLK_E, 1, EB)

    ppos = jnp.pad(pos.astype(f32), ((0, 0), (0, 125)))
    ppj = _sc_gather(ppos, j_s, 200)
    ppi = _sc_gather(ppos, i_s, 200)
    REPc = jnp.asarray(_REP_CONST)
    SELc = jnp.asarray(_SEL_CONST).astype(jnp.bfloat16)
    EXPc = jnp.asarray(_EXPAND_CONST).astype(jnp.bfloat16)

    x_pad = jnp.pad(x, ((0, NPAD - N), (0, 16 - NE)))
    Wemb_pad = jnp.pad(W_embed.astype(f32), ((0, 16 - NE), (0, 0)))

    S = jnp.asarray(_S_CONST)
    T = jnp.asarray(_TILE_CONST)
    PERM = jnp.asarray(_PERM_CONST)

    Wemb128 = jnp.pad(Wemb_pad, ((0, 0), (0, 128 - C)))
    h = _embed_call(x_pad, Wemb128)
    bat3 = jnp.pad(bat, (0, NPAD - N), constant_values=NG).reshape(
        NBLK_N, 1, NBN)

    layer_params = [
        (W1_0, W2_0, W3_0, mix_0, Wsc_0, velem_0, Wprod_0, Wout_0),
        (W1_1, W2_1, W3_1, mix_1, Wsc_1, velem_1, Wprod_1, Wout_1),
    ]

    nf = None
    s_table = h[:N]
    for li, (W1, W2, W3, mix, Wsc, velem, Wprod, Wout) in enumerate(
            layer_params):
        W3p = W3.astype(f32)[:, _W3PERM]
        BM = _big_blockdiag(mix.astype(f32), scale=1.0 / AVG_NEIGH)
        BS = _big_blockdiag(Wsc.astype(f32))
        BO = _big_blockdiag(Wout.astype(f32))
        ve = jnp.pad(velem.astype(f32), (0, 16 - NE)).reshape(16, 1)

        sj = _sc_gather(s_table, j_s, 200)
        agg = _edge_call(bases, ppj, ppi, ids3, sj, W1.astype(f32),
                         W2.astype(f32), W3p, REPc, SELc, EXPc)
        nf = _node_call(x_pad, agg, nf, Wemb_pad, BM, S,
                        Wprod.astype(f32), T, BO, BS, ve, first=(li == 0))
        s_table = jnp.pad(nf[:N, :C], ((0, 0), (0, 128 - C)))

    return _pool_call(nf, bat3, PERM)


# speedup vs baseline: 2.9202x; 1.0280x over previous
import functools

import jax
import jax.numpy as jnp
import numpy as np
from jax import lax
from jax.experimental import pallas as pl
from jax.experimental.pallas import tpu as pltpu
from jax.experimental.pallas import tpu_sc as plsc

N = 10000
E = 160000
NE = 10
C = 32
SH = 16
NB = 8
NG = 64
R_MAX = 5.0
AVG_NEIGH = 10.0
L_SLICES = [(0, 1), (1, 4), (4, 9), (9, 16)]
LOF = [0, 1, 1, 1, 2, 2, 2, 2, 2, 3, 3, 3, 3, 3, 3, 3]

EB = 1280
NBLK_E = E // EB
W = 128
NPAD = 11264
NBN = 1024
NBLK_N = NPAD // NBN

_INTERPRET = False


def _np_S():
    S = np.zeros((C * SH, C * 4), np.float32)
    for m in range(SH):
        l = LOF[m]
        for c in range(C):
            S[m * 32 + c, c * 4 + l] = 1.0
    return S


def _np_TILE():
    T = np.zeros((C, C * SH), np.float32)
    for m in range(SH):
        for c in range(C):
            T[c, m * 32 + c] = 1.0
    return T


def _np_PERM():
    P = np.zeros((C * SH, C * SH), np.float32)
    for m in range(SH):
        for c in range(C):
            P[m * 32 + c, c * 16 + m] = 1.0
    return P


_S_CONST = _np_S()
_TILE_CONST = _np_TILE()
_PERM_CONST = _np_PERM()
_W3PERM = np.array([[c * 4 + l for c in range(C)] for l in range(4)],
                   np.int32).reshape(-1)


def _np_REP():
    M = np.zeros((C, 128), np.float32)
    for l in range(4):
        for c in range(C):
            M[c, l * 32 + c] = 1.0
    return M


def _np_SEL():
    M = np.zeros((128, C * SH), np.float32)
    for m in range(SH):
        for c in range(C):
            M[LOF[m] * 32 + c, m * 32 + c] = 1.0
    return M


def _np_EXPAND():
    M = np.zeros((SH, C * SH), np.float32)
    for m in range(SH):
        for c in range(C):
            M[m, m * 32 + c] = 1.0
    return M


_REP_CONST = _np_REP()
_SEL_CONST = _np_SEL().astype(np.float32)
_EXPAND_CONST = _np_EXPAND()


def _sph_harm_cols(x, y, z):
    s3 = np.sqrt(3.0); s15 = np.sqrt(15.0); s5 = np.sqrt(5.0)
    c1 = np.sqrt(35.0 / 8.0); c2 = np.sqrt(105.0); c3 = np.sqrt(21.0 / 8.0)
    c4 = np.sqrt(7.0) / 2.0; c5 = np.sqrt(105.0) / 2.0
    one = jnp.ones_like(x)
    return [
        one,
        s3 * x, s3 * y, s3 * z,
        s15 * x * y, s15 * y * z, (s5 / 2.0) * (3.0 * z * z - one),
        s15 * x * z, (s15 / 2.0) * (x * x - y * y),
        c1 * y * (3.0 * x * x - y * y), c2 * x * y * z,
        c3 * y * (5.0 * z * z - one), c4 * z * (5.0 * z * z - 3.0 * one),
        c3 * x * (5.0 * z * z - one), c5 * z * (x * x - y * y),
        c1 * x * (x * x - 3.0 * y * y),
    ]


def _edge_kernel(bases_ref, pj_ref, pi_ref, ids_ref, sj_ref, W1_ref, W2_ref,
                 W3p_ref, REP_ref, SEL_ref, EXP_ref, out_ref, acc_ref):
    b = pl.program_id(0)

    @pl.when(b == 0)
    def _():
        acc_ref[...] = jnp.zeros_like(acc_ref)

    @pl.when(b >= NBLK_E)
    def _():
        off = pl.multiple_of((b - NBLK_E) * NBN, NBN)
        out_ref[...] = acc_ref[pl.ds(off, NBN), :]

    @pl.when(b < NBLK_E)
    def _():
        _edge_block(bases_ref, pj_ref, pi_ref, ids_ref, sj_ref, W1_ref,
                    W2_ref, W3p_ref, REP_ref, SEL_ref, EXP_ref, acc_ref, b)


def _edge_block(bases_ref, pj_ref, pi_ref, ids_ref, sj_ref, W1_ref, W2_ref,
                W3p_ref, REP_ref, SEL_ref, EXP_ref, acc_ref, b):
    def mm(a, bm, pt=jnp.float32):
        return jax.lax.dot_general(a, bm, (((1,), (0,)), ((), ())),
                                   preferred_element_type=pt)

    dT = jnp.transpose(pj_ref[:, :16] - pi_ref[:, :16])
    vx = dT[0:1, :]
    vy = dT[1:2, :]
    vz = dT[2:3, :]
    r2 = vx * vx + vy * vy + vz * vz
    r = jnp.sqrt(r2)
    rc = jnp.maximum(r, 1e-9)
    rinv = 1.0 / rc
    ux = vx * rinv; uy = vy * rinv; uz = vz * rinv
    YT = jnp.concatenate(_sph_harm_cols(ux, uy, uz), axis=0)

    p = 5.0
    ur = r * np.float32(1.0 / R_MAX)
    u5 = ur * ur * ur * ur * ur
    cut = (1.0 - 0.5 * (p + 1.0) * (p + 2.0) * u5 + p * (p + 2.0) * u5 * ur
           - 0.5 * p * (p + 1.0) * u5 * ur * ur)
    cut = cut * (ur < 1.0).astype(jnp.float32)
    scal = np.float32(np.sqrt(2.0 / R_MAX))
    amp = scal * rinv * cut
    efT = jnp.concatenate(
        [jnp.sin(np.float32(n * np.pi / R_MAX) * r) * amp
         for n in range(1, NB + 1)], axis=0)

    ef = jnp.transpose(efT)
    Y = jnp.transpose(YT)

    h1 = jax.nn.silu(mm(ef, W1_ref[...]))
    h2 = jax.nn.silu(mm(h1, W2_ref[...]))
    R2 = mm(h2, W3p_ref[...])

    sj = sj_ref[:, :C]
    T = R2 * mm(sj, REP_ref[...])
    Tsel = mm(T.astype(jnp.bfloat16), SEL_ref[...]).astype(jnp.bfloat16)
    Yexp = mm(Y.astype(jnp.bfloat16), EXP_ref[...]).astype(jnp.bfloat16)
    msg = Tsel * Yexp

    ids = ids_ref[0]
    base = pl.multiple_of(bases_ref[b], 8)
    rel = jnp.broadcast_to(ids - base, (W, EB))
    iot = jax.lax.broadcasted_iota(jnp.int32, (W, EB), 0)
    onehotT = (rel == iot).astype(jnp.bfloat16)

    contrib = jax.lax.dot_general(
        onehotT, msg, (((1,), (0,)), ((), ())),
        preferred_element_type=jnp.float32)
    cur = acc_ref[pl.ds(base, W), :]
    acc_ref[pl.ds(base, W), :] = cur + contrib


def _edge_call(bases, ppj, ppi, ids3, sj, W1, W2, W3p, REPc, SELc, EXPc):
    cap = lambda b: jnp.minimum(b, NBLK_E - 1)
    spec = pltpu.PrefetchScalarGridSpec(
        num_scalar_prefetch=1,
        grid=(NBLK_E + NBLK_N,),
        in_specs=[
            pl.BlockSpec((EB, 128), lambda b, s: (cap(b), 0)),
            pl.BlockSpec((EB, 128), lambda b, s: (cap(b), 0)),
            pl.BlockSpec((1, 1, EB), lambda b, s: (cap(b), 0, 0)),
            pl.BlockSpec((EB, 128), lambda b, s: (cap(b), 0)),
            pl.BlockSpec((NB, 64), lambda b, s: (0, 0)),
            pl.BlockSpec((64, 64), lambda b, s: (0, 0)),
            pl.BlockSpec((64, 128), lambda b, s: (0, 0)),
            pl.BlockSpec((C, 128), lambda b, s: (0, 0)),
            pl.BlockSpec((128, C * SH), lambda b, s: (0, 0)),
            pl.BlockSpec((SH, C * SH), lambda b, s: (0, 0)),
        ],
        out_specs=pl.BlockSpec(
            (NBN, C * SH),
            lambda b, s: (jnp.maximum(b - NBLK_E, 0), 0)),
        scratch_shapes=[pltpu.VMEM((NPAD, C * SH), jnp.float32)],
    )
    return pl.pallas_call(
        _edge_kernel,
        grid_spec=spec,
        out_shape=jax.ShapeDtypeStruct((NPAD, C * SH), jnp.float32),
        compiler_params=pltpu.CompilerParams(
            dimension_semantics=("arbitrary",)),
        interpret=_INTERPRET,
    )(bases, ppj, ppi, ids3, sj, W1, W2, W3p, REPc, SELc, EXPc)


def _node_kernel_first(x_ref, agg_ref, Wemb_ref, BM_ref, S_ref, Wp_ref,
                       T_ref, BO_ref, BS_ref, ve_ref, nf_ref):
    _node_body(x_ref, agg_ref, None, Wemb_ref, BM_ref, S_ref, Wp_ref, T_ref,
               BO_ref, BS_ref, ve_ref, nf_ref, first=True)


def _node_kernel_rest(x_ref, agg_ref, nfin_ref, BM_ref, S_ref, Wp_ref,
                      T_ref, BO_ref, BS_ref, ve_ref, nf_ref):
    _node_body(x_ref, agg_ref, nfin_ref, None, BM_ref, S_ref, Wp_ref, T_ref,
               BO_ref, BS_ref, ve_ref, nf_ref, first=False)


def _node_body(x_ref, agg_ref, nfin_ref, Wemb_ref, BM_ref, S_ref, Wp_ref,
               T_ref, BO_ref, BS_ref, ve_ref, nf_ref, first):
    x = x_ref[...]
    agg = agg_ref[...]

    def mm(a, b):
        return jax.lax.dot_general(a, b, (((1,), (0,)), ((), ())),
                                   preferred_element_type=jnp.float32)

    if first:
        h = mm(x, Wemb_ref[...])
        nf_cur = jnp.concatenate(
            [h, jnp.zeros((NBN, C * SH - C), jnp.float32)], axis=1)
    else:
        nf_cur = nfin_ref[...]

    mid = mm(agg, BM_ref[...])
    inv = mm(mid * mid, S_ref[...])
    g = jax.nn.silu(mm(inv, Wp_ref[...]))
    gt = mm(g, T_ref[...])
    ew = mm(x, ve_ref[...])
    sc = mm(nf_cur, BS_ref[...]) * ew
    nf_ref[...] = mm(mid, BO_ref[...]) * gt + sc


def _node_call(x, agg, nfin, Wemb, BM, S, Wp, T, BO, BS, ve, first):
    full = lambda shape: pl.BlockSpec(shape, lambda b: (0, 0))
    in_specs = [
        pl.BlockSpec((NBN, 16), lambda b: (b, 0)),
        pl.BlockSpec((NBN, C * SH), lambda b: (b, 0)),
    ]
    args = [x, agg]
    if first:
        kern = _node_kernel_first
        in_specs.append(full((16, C)))
        args.append(Wemb)
    else:
        kern = _node_kernel_rest
        in_specs.append(pl.BlockSpec((NBN, C * SH), lambda b: (b, 0)))
        args.append(nfin)
    in_specs += [full((C * SH, C * SH)), full((C * SH, C * 4)),
                 full((C * 4, C)), full((C, C * SH)),
                 full((C * SH, C * SH)), full((C * SH, C * SH)),
                 full((16, 1))]
    args += [BM, S, Wp, T, BO, BS, ve]
    return pl.pallas_call(
        kern,
        grid=(NBLK_N,),
        in_specs=in_specs,
        out_specs=pl.BlockSpec((NBN, C * SH), lambda b: (b, 0)),
        out_shape=jax.ShapeDtypeStruct((NPAD, C * SH), jnp.float32),
        compiler_params=pltpu.CompilerParams(
            dimension_semantics=("arbitrary",)),
        interpret=_INTERPRET,
    )(*args)


def _pool_kernel(nf_ref, bat_ref, PERM_ref, out_ref, acc, cnt):
    b = pl.program_id(0)

    @pl.when(b == 0)
    def _():
        acc[...] = jnp.zeros_like(acc)
        cnt[...] = jnp.zeros_like(cnt)

    bat = bat_ref[0]
    iot = jax.lax.broadcasted_iota(jnp.int32, (NG, NBN), 0)
    oh = (jnp.broadcast_to(bat, (NG, NBN)) == iot).astype(jnp.bfloat16)
    nfb = nf_ref[...].astype(jnp.bfloat16)

    def mm(a, b_, pt=jnp.float32):
        return jax.lax.dot_general(a, b_, (((1,), (0,)), ((), ())),
                                   preferred_element_type=pt)

    acc[...] = acc[...] + mm(oh, nfb)
    ones = jnp.ones((NBN, 128), jnp.bfloat16)
    cnt[...] = cnt[...] + mm(oh, ones)

    @pl.when(b == NBLK_N - 1)
    def _():
        c = jnp.maximum(cnt[:, 0:1], 1.0)
        out_ref[...] = mm(acc[...] / c, PERM_ref[...])


def _pool_call(nf, bat3, PERM):
    return pl.pallas_call(
        _pool_kernel,
        grid=(NBLK_N,),
        in_specs=[
            pl.BlockSpec((NBN, C * SH), lambda b: (b, 0)),
            pl.BlockSpec((1, 1, NBN), lambda b: (b, 0, 0)),
            pl.BlockSpec((C * SH, C * SH), lambda b: (0, 0)),
        ],
        out_specs=pl.BlockSpec((NG, C * SH), lambda b: (0, 0)),
        out_shape=jax.ShapeDtypeStruct((NG, C * SH), jnp.float32),
        scratch_shapes=[pltpu.VMEM((NG, C * SH), jnp.float32),
                        pltpu.VMEM((NG, 128), jnp.float32)],
        compiler_params=pltpu.CompilerParams(
            dimension_semantics=("arbitrary",)),
        interpret=_INTERPRET,
    )(nf, bat3, PERM)


def _sc_gather(table, idx, chunk):
    V, D = table.shape
    B = idx.shape[0]
    NW = 32
    b_per_w = B // NW
    nchunk = b_per_w // chunk
    mesh = plsc.VectorSubcoreMesh(core_axis_name="c", subcore_axis_name="s")

    @functools.partial(
        pl.kernel, mesh=mesh,
        out_type=jax.ShapeDtypeStruct((B, D), jnp.float32),
        compiler_params=pltpu.CompilerParams(use_tc_tiling_on_sc=True),
        scratch_types=[
            pltpu.VMEM((chunk,), jnp.int32),
            pltpu.VMEM((chunk, D), jnp.float32),
            pltpu.SemaphoreType.DMA,
        ],
    )
    def k(table_hbm, idx_hbm, out_hbm, idx_v, rows_v, sem):
        wid = lax.axis_index("s") * 2 + lax.axis_index("c")
        for ci in range(nchunk):
            base = wid * b_per_w + ci * chunk
            pltpu.sync_copy(idx_hbm.at[pl.ds(base, chunk)], idx_v)
            pltpu.async_copy(table_hbm.at[idx_v], rows_v, sem).wait()
            pltpu.sync_copy(rows_v, out_hbm.at[pl.ds(base, chunk)])

    return k(table, idx)


def _embed_kernel(x_ref, W_ref, h_ref):
    h_ref[...] = jax.lax.dot_general(
        x_ref[...], W_ref[...], (((1,), (0,)), ((), ())),
        preferred_element_type=jnp.float32)


def _embed_call(x_pad, Wemb128):
    return pl.pallas_call(
        _embed_kernel,
        out_shape=jax.ShapeDtypeStruct((NPAD, 128), jnp.float32),
        interpret=_INTERPRET,
    )(x_pad, Wemb128)


def _big_blockdiag(w4, scale=1.0):
    z = jnp.zeros((C, C), jnp.float32)
    return jnp.concatenate(
        [jnp.concatenate([z] * m + [w4[LOF[m]] * scale]
                         + [z] * (SH - 1 - m), axis=1)
         for m in range(SH)], axis=0)


def kernel(x, pos, batch, edge_index, W_embed, W1_0, W2_0, W3_0, mix_0,
           Wsc_0, velem_0, Wprod_0, Wout_0, W1_1, W2_1, W3_1, mix_1, Wsc_1,
           velem_1, Wprod_1, Wout_1):
    f32 = jnp.float32
    x = x.astype(f32)
    j = edge_index[0].astype(jnp.int32)
    i = edge_index[1].astype(jnp.int32)
    bat = batch.astype(jnp.int32)

    order = jnp.argsort(i)
    i_s = i[order]
    j_s = j[order]
    bases = ((i_s[::EB] // 8) * 8).astype(jnp.int32)
    ids3 = i_s.reshape(NBLK_E, 1, EB)

    ppos = jnp.pad(pos.astype(f32), ((0, 0), (0, 125)))
    ppj = _sc_gather(ppos, j_s, 200)
    ppi = _sc_gather(ppos, i_s, 200)
    REPc = jnp.asarray(_REP_CONST)
    SELc = jnp.asarray(_SEL_CONST).astype(jnp.bfloat16)
    EXPc = jnp.asarray(_EXPAND_CONST).astype(jnp.bfloat16)

    x_pad = jnp.pad(x, ((0, NPAD - N), (0, 16 - NE)))
    Wemb_pad = jnp.pad(W_embed.astype(f32), ((0, 16 - NE), (0, 0)))

    S = jnp.asarray(_S_CONST)
    T = jnp.asarray(_TILE_CONST)
    PERM = jnp.asarray(_PERM_CONST)

    Wemb128 = jnp.pad(Wemb_pad, ((0, 0), (0, 128 - C)))
    h = _embed_call(x_pad, Wemb128)
    bat3 = jnp.pad(bat, (0, NPAD - N), constant_values=NG).reshape(
        NBLK_N, 1, NBN)

    layer_params = [
        (W1_0, W2_0, W3_0, mix_0, Wsc_0, velem_0, Wprod_0, Wout_0),
        (W1_1, W2_1, W3_1, mix_1, Wsc_1, velem_1, Wprod_1, Wout_1),
    ]

    nf = None
    s_table = h[:N]
    for li, (W1, W2, W3, mix, Wsc, velem, Wprod, Wout) in enumerate(
            layer_params):
        W3p = W3.astype(f32)[:, _W3PERM]
        BM = _big_blockdiag(mix.astype(f32), scale=1.0 / AVG_NEIGH)
        BS = _big_blockdiag(Wsc.astype(f32))
        BO = _big_blockdiag(Wout.astype(f32))
        ve = jnp.pad(velem.astype(f32), (0, 16 - NE)).reshape(16, 1)

        sj = _sc_gather(s_table, j_s, 200)
        agg = _edge_call(bases, ppj, ppi, ids3, sj, W1.astype(f32),
                         W2.astype(f32), W3p, REPc, SELc, EXPc)
        nf = _node_call(x_pad, agg, nf, Wemb_pad, BM, S,
                        Wprod.astype(f32), T, BO, BS, ve, first=(li == 0))
        s_table = jnp.pad(nf[:N, :C], ((0, 0), (0, 128 - C)))

    return _pool_call(nf, bat3, PERM)


# speedup vs baseline: 3.3951x; 1.1626x over previous
import functools

import jax
import jax.numpy as jnp
import numpy as np
from jax import lax
from jax.experimental import pallas as pl
from jax.experimental.pallas import tpu as pltpu
from jax.experimental.pallas import tpu_sc as plsc

N = 10000
E = 160000
NE = 10
C = 32
SH = 16
NB = 8
NG = 64
R_MAX = 5.0
AVG_NEIGH = 10.0
L_SLICES = [(0, 1), (1, 4), (4, 9), (9, 16)]
LOF = [0, 1, 1, 1, 2, 2, 2, 2, 2, 3, 3, 3, 3, 3, 3, 3]

EB = 1280
NBLK_E = E // EB
W = 128
NPAD = 11264
NBN = 1024
NBLK_N = NPAD // NBN

_INTERPRET = False


def _np_S():
    S = np.zeros((C * SH, C * 4), np.float32)
    for m in range(SH):
        l = LOF[m]
        for c in range(C):
            S[m * 32 + c, c * 4 + l] = 1.0
    return S


def _np_TILE():
    T = np.zeros((C, C * SH), np.float32)
    for m in range(SH):
        for c in range(C):
            T[c, m * 32 + c] = 1.0
    return T


def _np_PERM():
    P = np.zeros((C * SH, C * SH), np.float32)
    for m in range(SH):
        for c in range(C):
            P[m * 32 + c, c * 16 + m] = 1.0
    return P


_S_CONST = _np_S()
_TILE_CONST = _np_TILE()
_PERM_CONST = _np_PERM()
_W3PERM = np.array([[c * 4 + l for c in range(C)] for l in range(4)],
                   np.int32).reshape(-1)


def _np_REP():
    M = np.zeros((C, 128), np.float32)
    for l in range(4):
        for c in range(C):
            M[c, l * 32 + c] = 1.0
    return M


def _np_SEL():
    M = np.zeros((128, C * SH), np.float32)
    for m in range(SH):
        for c in range(C):
            M[LOF[m] * 32 + c, m * 32 + c] = 1.0
    return M


def _np_EXPAND():
    M = np.zeros((SH, C * SH), np.float32)
    for m in range(SH):
        for c in range(C):
            M[m, m * 32 + c] = 1.0
    return M


_REP_CONST = _np_REP()
_SEL_CONST = _np_SEL().astype(np.float32)
_EXPAND_CONST = _np_EXPAND()


def _sph_harm_cols(x, y, z):
    s3 = np.sqrt(3.0); s15 = np.sqrt(15.0); s5 = np.sqrt(5.0)
    c1 = np.sqrt(35.0 / 8.0); c2 = np.sqrt(105.0); c3 = np.sqrt(21.0 / 8.0)
    c4 = np.sqrt(7.0) / 2.0; c5 = np.sqrt(105.0) / 2.0
    one = jnp.ones_like(x)
    return [
        one,
        s3 * x, s3 * y, s3 * z,
        s15 * x * y, s15 * y * z, (s5 / 2.0) * (3.0 * z * z - one),
        s15 * x * z, (s15 / 2.0) * (x * x - y * y),
        c1 * y * (3.0 * x * x - y * y), c2 * x * y * z,
        c3 * y * (5.0 * z * z - one), c4 * z * (5.0 * z * z - 3.0 * one),
        c3 * x * (5.0 * z * z - one), c5 * z * (x * x - y * y),
        c1 * x * (x * x - 3.0 * y * y),
    ]


def _edge_kernel(bases_ref, geom_ref, ids_ref, sj_ref, W1_ref, W2_ref,
                 W3p_ref, REP_ref, SEL_ref, EXP_ref, out_ref, acc_ref):
    b = pl.program_id(0)

    @pl.when(b == 0)
    def _():
        acc_ref[...] = jnp.zeros_like(acc_ref)

    @pl.when(b >= NBLK_E)
    def _():
        off = pl.multiple_of((b - NBLK_E) * NBN, NBN)
        out_ref[...] = acc_ref[pl.ds(off, NBN), :]

    @pl.when(b < NBLK_E)
    def _():
        _edge_block(bases_ref, geom_ref, ids_ref, sj_ref, W1_ref,
                    W2_ref, W3p_ref, REP_ref, SEL_ref, EXP_ref, acc_ref, b)


def _geom_kernel(pj_ref, pi_ref, out_ref):
    dT = jnp.transpose(pj_ref[:, :16] - pi_ref[:, :16])
    vx = dT[0:1, :]
    vy = dT[1:2, :]
    vz = dT[2:3, :]
    r2 = vx * vx + vy * vy + vz * vz
    r = jnp.sqrt(r2)
    rc = jnp.maximum(r, 1e-9)
    rinv = 1.0 / rc
    ux = vx * rinv; uy = vy * rinv; uz = vz * rinv
    YT = jnp.concatenate(_sph_harm_cols(ux, uy, uz), axis=0)

    p = 5.0
    ur = r * np.float32(1.0 / R_MAX)
    u5 = ur * ur * ur * ur * ur
    cut = (1.0 - 0.5 * (p + 1.0) * (p + 2.0) * u5 + p * (p + 2.0) * u5 * ur
           - 0.5 * p * (p + 1.0) * u5 * ur * ur)
    cut = cut * (ur < 1.0).astype(jnp.float32)
    scal = np.float32(np.sqrt(2.0 / R_MAX))
    amp = scal * rinv * cut
    theta = np.float32(np.pi / R_MAX) * r
    s1 = jnp.sin(theta)
    c2 = 2.0 * jnp.cos(theta)
    rows = [s1]
    prev2, prev1 = jnp.zeros_like(s1), s1
    for _ in range(NB - 1):
        cur = c2 * prev1 - prev2
        rows.append(cur)
        prev2, prev1 = prev1, cur
    efT = jnp.concatenate([rw * amp for rw in rows], axis=0)

    gT = jnp.concatenate([YT, efT], axis=0)
    out_ref[:, :24] = jnp.transpose(gT)
    out_ref[:, 24:] = jnp.zeros((EB, 104), jnp.float32)


def _geom_call(ppj, ppi):
    return pl.pallas_call(
        _geom_kernel,
        grid=(NBLK_E,),
        in_specs=[
            pl.BlockSpec((EB, 128), lambda b: (b, 0)),
            pl.BlockSpec((EB, 128), lambda b: (b, 0)),
        ],
        out_specs=pl.BlockSpec((EB, 128), lambda b: (b, 0)),
        out_shape=jax.ShapeDtypeStruct((E, 128), jnp.float32),
        compiler_params=pltpu.CompilerParams(
            dimension_semantics=("arbitrary",)),
        interpret=_INTERPRET,
    )(ppj, ppi)


def _edge_block(bases_ref, geom_ref, ids_ref, sj_ref, W1_ref, W2_ref,
                W3p_ref, REP_ref, SEL_ref, EXP_ref, acc_ref, b):
    def mm(a, bm, pt=jnp.float32):
        return jax.lax.dot_general(a, bm, (((1,), (0,)), ((), ())),
                                   preferred_element_type=pt)

    Y = geom_ref[:, :SH]
    ef = geom_ref[:, SH:SH + NB]

    h1 = jax.nn.silu(mm(ef, W1_ref[...]))
    h2 = jax.nn.silu(mm(h1, W2_ref[...]))
    R2 = mm(h2, W3p_ref[...])

    sj = sj_ref[:, :C]
    T = R2 * mm(sj, REP_ref[...])
    Tsel = mm(T.astype(jnp.bfloat16), SEL_ref[...]).astype(jnp.bfloat16)
    Yexp = mm(Y.astype(jnp.bfloat16), EXP_ref[...]).astype(jnp.bfloat16)
    msg = Tsel * Yexp

    ids = ids_ref[0]
    base = pl.multiple_of(bases_ref[b], 8)
    rel = jnp.broadcast_to(ids - base, (W, EB))
    iot = jax.lax.broadcasted_iota(jnp.int32, (W, EB), 0)
    onehotT = (rel == iot).astype(jnp.bfloat16)

    contrib = jax.lax.dot_general(
        onehotT, msg, (((1,), (0,)), ((), ())),
        preferred_element_type=jnp.float32)
    cur = acc_ref[pl.ds(base, W), :]
    acc_ref[pl.ds(base, W), :] = cur + contrib


def _edge_call(bases, geom, ids3, sj, W1, W2, W3p, REPc, SELc, EXPc):
    cap = lambda b: jnp.minimum(b, NBLK_E - 1)
    spec = pltpu.PrefetchScalarGridSpec(
        num_scalar_prefetch=1,
        grid=(NBLK_E + NBLK_N,),
        in_specs=[
            pl.BlockSpec((EB, 128), lambda b, s: (cap(b), 0)),
            pl.BlockSpec((1, 1, EB), lambda b, s: (cap(b), 0, 0)),
            pl.BlockSpec((EB, 128), lambda b, s: (cap(b), 0)),
            pl.BlockSpec((NB, 64), lambda b, s: (0, 0)),
            pl.BlockSpec((64, 64), lambda b, s: (0, 0)),
            pl.BlockSpec((64, 128), lambda b, s: (0, 0)),
            pl.BlockSpec((C, 128), lambda b, s: (0, 0)),
            pl.BlockSpec((128, C * SH), lambda b, s: (0, 0)),
            pl.BlockSpec((SH, C * SH), lambda b, s: (0, 0)),
        ],
        out_specs=pl.BlockSpec(
            (NBN, C * SH),
            lambda b, s: (jnp.maximum(b - NBLK_E, 0), 0)),
        scratch_shapes=[pltpu.VMEM((NPAD, C * SH), jnp.float32)],
    )
    return pl.pallas_call(
        _edge_kernel,
        grid_spec=spec,
        out_shape=jax.ShapeDtypeStruct((NPAD, C * SH), jnp.float32),
        compiler_params=pltpu.CompilerParams(
            dimension_semantics=("arbitrary",)),
        interpret=_INTERPRET,
    )(bases, geom, ids3, sj, W1, W2, W3p, REPc, SELc, EXPc)


def _node_kernel_first(x_ref, agg_ref, Wemb_ref, BM_ref, S_ref, Wp_ref,
                       T_ref, BO_ref, BS_ref, ve_ref, nf_ref):
    _node_body(x_ref, agg_ref, None, Wemb_ref, BM_ref, S_ref, Wp_ref, T_ref,
               BO_ref, BS_ref, ve_ref, nf_ref, first=True)


def _node_kernel_rest(x_ref, agg_ref, nfin_ref, BM_ref, S_ref, Wp_ref,
                      T_ref, BO_ref, BS_ref, ve_ref, nf_ref):
    _node_body(x_ref, agg_ref, nfin_ref, None, BM_ref, S_ref, Wp_ref, T_ref,
               BO_ref, BS_ref, ve_ref, nf_ref, first=False)


def _node_body(x_ref, agg_ref, nfin_ref, Wemb_ref, BM_ref, S_ref, Wp_ref,
               T_ref, BO_ref, BS_ref, ve_ref, nf_ref, first):
    x = x_ref[...]
    agg = agg_ref[...]

    def mm(a, b):
        return jax.lax.dot_general(a, b, (((1,), (0,)), ((), ())),
                                   preferred_element_type=jnp.float32)

    if first:
        h = mm(x, Wemb_ref[...])
        nf_cur = jnp.concatenate(
            [h, jnp.zeros((NBN, C * SH - C), jnp.float32)], axis=1)
    else:
        nf_cur = nfin_ref[...]

    mid = mm(agg, BM_ref[...])
    inv = mm(mid * mid, S_ref[...])
    g = jax.nn.silu(mm(inv, Wp_ref[...]))
    gt = mm(g, T_ref[...])
    ew = mm(x, ve_ref[...])
    sc = mm(nf_cur, BS_ref[...]) * ew
    nf_ref[...] = mm(mid, BO_ref[...]) * gt + sc


def _node_call(x, agg, nfin, Wemb, BM, S, Wp, T, BO, BS, ve, first):
    full = lambda shape: pl.BlockSpec(shape, lambda b: (0, 0))
    in_specs = [
        pl.BlockSpec((NBN, 16), lambda b: (b, 0)),
        pl.BlockSpec((NBN, C * SH), lambda b: (b, 0)),
    ]
    args = [x, agg]
    if first:
        kern = _node_kernel_first
        in_specs.append(full((16, C)))
        args.append(Wemb)
    else:
        kern = _node_kernel_rest
        in_specs.append(pl.BlockSpec((NBN, C * SH), lambda b: (b, 0)))
        args.append(nfin)
    in_specs += [full((C * SH, C * SH)), full((C * SH, C * 4)),
                 full((C * 4, C)), full((C, C * SH)),
                 full((C * SH, C * SH)), full((C * SH, C * SH)),
                 full((16, 1))]
    args += [BM, S, Wp, T, BO, BS, ve]
    return pl.pallas_call(
        kern,
        grid=(NBLK_N,),
        in_specs=in_specs,
        out_specs=pl.BlockSpec((NBN, C * SH), lambda b: (b, 0)),
        out_shape=jax.ShapeDtypeStruct((NPAD, C * SH), jnp.float32),
        compiler_params=pltpu.CompilerParams(
            dimension_semantics=("arbitrary",)),
        interpret=_INTERPRET,
    )(*args)


def _pool_kernel(nf_ref, bat_ref, PERM_ref, out_ref, acc, cnt):
    b = pl.program_id(0)

    @pl.when(b == 0)
    def _():
        acc[...] = jnp.zeros_like(acc)
        cnt[...] = jnp.zeros_like(cnt)

    bat = bat_ref[0]
    iot = jax.lax.broadcasted_iota(jnp.int32, (NG, NBN), 0)
    oh = (jnp.broadcast_to(bat, (NG, NBN)) == iot).astype(jnp.bfloat16)
    nfb = nf_ref[...].astype(jnp.bfloat16)

    def mm(a, b_, pt=jnp.float32):
        return jax.lax.dot_general(a, b_, (((1,), (0,)), ((), ())),
                                   preferred_element_type=pt)

    acc[...] = acc[...] + mm(oh, nfb)
    ones = jnp.ones((NBN, 128), jnp.bfloat16)
    cnt[...] = cnt[...] + mm(oh, ones)

    @pl.when(b == NBLK_N - 1)
    def _():
        c = jnp.maximum(cnt[:, 0:1], 1.0)
        out_ref[...] = mm(acc[...] / c, PERM_ref[...])


def _pool_call(nf, bat3, PERM):
    return pl.pallas_call(
        _pool_kernel,
        grid=(NBLK_N,),
        in_specs=[
            pl.BlockSpec((NBN, C * SH), lambda b: (b, 0)),
            pl.BlockSpec((1, 1, NBN), lambda b: (b, 0, 0)),
            pl.BlockSpec((C * SH, C * SH), lambda b: (0, 0)),
        ],
        out_specs=pl.BlockSpec((NG, C * SH), lambda b: (0, 0)),
        out_shape=jax.ShapeDtypeStruct((NG, C * SH), jnp.float32),
        scratch_shapes=[pltpu.VMEM((NG, C * SH), jnp.float32),
                        pltpu.VMEM((NG, 128), jnp.float32)],
        compiler_params=pltpu.CompilerParams(
            dimension_semantics=("arbitrary",)),
        interpret=_INTERPRET,
    )(nf, bat3, PERM)


def _sc_gather(table, idx, chunk):
    V, D = table.shape
    B = idx.shape[0]
    NW = 32
    b_per_w = B // NW
    nchunk = b_per_w // chunk
    mesh = plsc.VectorSubcoreMesh(core_axis_name="c", subcore_axis_name="s")

    @functools.partial(
        pl.kernel, mesh=mesh,
        out_type=jax.ShapeDtypeStruct((B, D), jnp.float32),
        compiler_params=pltpu.CompilerParams(use_tc_tiling_on_sc=True),
        scratch_types=[
            pltpu.VMEM((chunk,), jnp.int32),
            pltpu.VMEM((chunk, D), jnp.float32),
            pltpu.SemaphoreType.DMA,
        ],
    )
    def k(table_hbm, idx_hbm, out_hbm, idx_v, rows_v, sem):
        wid = lax.axis_index("s") * 2 + lax.axis_index("c")
        for ci in range(nchunk):
            base = wid * b_per_w + ci * chunk
            pltpu.sync_copy(idx_hbm.at[pl.ds(base, chunk)], idx_v)
            pltpu.async_copy(table_hbm.at[idx_v], rows_v, sem).wait()
            pltpu.sync_copy(rows_v, out_hbm.at[pl.ds(base, chunk)])

    return k(table, idx)


def _embed_kernel(x_ref, W_ref, h_ref):
    h_ref[...] = jax.lax.dot_general(
        x_ref[...], W_ref[...], (((1,), (0,)), ((), ())),
        preferred_element_type=jnp.float32)


def _embed_call(x_pad, Wemb128):
    return pl.pallas_call(
        _embed_kernel,
        out_shape=jax.ShapeDtypeStruct((NPAD, 128), jnp.float32),
        interpret=_INTERPRET,
    )(x_pad, Wemb128)


def _big_blockdiag(w4, scale=1.0):
    z = jnp.zeros((C, C), jnp.float32)
    return jnp.concatenate(
        [jnp.concatenate([z] * m + [w4[LOF[m]] * scale]
                         + [z] * (SH - 1 - m), axis=1)
         for m in range(SH)], axis=0)


def kernel(x, pos, batch, edge_index, W_embed, W1_0, W2_0, W3_0, mix_0,
           Wsc_0, velem_0, Wprod_0, Wout_0, W1_1, W2_1, W3_1, mix_1, Wsc_1,
           velem_1, Wprod_1, Wout_1):
    f32 = jnp.float32
    x = x.astype(f32)
    j = edge_index[0].astype(jnp.int32)
    i = edge_index[1].astype(jnp.int32)
    bat = batch.astype(jnp.int32)

    order = jnp.argsort(i)
    i_s = i[order]
    j_s = j[order]
    bases = ((i_s[::EB] // 8) * 8).astype(jnp.int32)
    ids3 = i_s.reshape(NBLK_E, 1, EB)

    ppos = jnp.pad(pos.astype(f32), ((0, 0), (0, 125)))
    ppj = _sc_gather(ppos, j_s, 200)
    ppi = _sc_gather(ppos, i_s, 200)
    geom = _geom_call(ppj, ppi)
    REPc = jnp.asarray(_REP_CONST)
    SELc = jnp.asarray(_SEL_CONST).astype(jnp.bfloat16)
    EXPc = jnp.asarray(_EXPAND_CONST).astype(jnp.bfloat16)

    x_pad = jnp.pad(x, ((0, NPAD - N), (0, 16 - NE)))
    Wemb_pad = jnp.pad(W_embed.astype(f32), ((0, 16 - NE), (0, 0)))

    S = jnp.asarray(_S_CONST)
    T = jnp.asarray(_TILE_CONST)
    PERM = jnp.asarray(_PERM_CONST)

    Wemb128 = jnp.pad(Wemb_pad, ((0, 0), (0, 128 - C)))
    h = _embed_call(x_pad, Wemb128)
    bat3 = jnp.pad(bat, (0, NPAD - N), constant_values=NG).reshape(
        NBLK_N, 1, NBN)

    layer_params = [
        (W1_0, W2_0, W3_0, mix_0, Wsc_0, velem_0, Wprod_0, Wout_0),
        (W1_1, W2_1, W3_1, mix_1, Wsc_1, velem_1, Wprod_1, Wout_1),
    ]

    nf = None
    s_table = h[:N]
    for li, (W1, W2, W3, mix, Wsc, velem, Wprod, Wout) in enumerate(
            layer_params):
        W3p = W3.astype(f32)[:, _W3PERM]
        BM = _big_blockdiag(mix.astype(f32), scale=1.0 / AVG_NEIGH)
        BS = _big_blockdiag(Wsc.astype(f32))
        BO = _big_blockdiag(Wout.astype(f32))
        ve = jnp.pad(velem.astype(f32), (0, 16 - NE)).reshape(16, 1)

        sj = _sc_gather(s_table, j_s, 200)
        agg = _edge_call(bases, geom, ids3, sj, W1.astype(f32),
                         W2.astype(f32), W3p, REPc, SELc, EXPc)
        nf = _node_call(x_pad, agg, nf, Wemb_pad, BM, S,
                        Wprod.astype(f32), T, BO, BS, ve, first=(li == 0))
        s_table = jnp.pad(nf[:N, :C], ((0, 0), (0, 128 - C)))

    return _pool_call(nf, bat3, PERM)


# speedup vs baseline: 3.4523x; 1.0169x over previous
import functools

import jax
import jax.numpy as jnp
import numpy as np
from jax import lax
from jax.experimental import pallas as pl
from jax.experimental.pallas import tpu as pltpu
from jax.experimental.pallas import tpu_sc as plsc

N = 10000
E = 160000
NE = 10
C = 32
SH = 16
NB = 8
NG = 64
R_MAX = 5.0
AVG_NEIGH = 10.0
L_SLICES = [(0, 1), (1, 4), (4, 9), (9, 16)]
LOF = [0, 1, 1, 1, 2, 2, 2, 2, 2, 3, 3, 3, 3, 3, 3, 3]

EB = 3200
NBLK_E = E // EB
W = 256
NPAD = 11264
NBN = 1024
NBLK_N = NPAD // NBN

_INTERPRET = False


def _np_S():
    S = np.zeros((C * SH, C * 4), np.float32)
    for m in range(SH):
        l = LOF[m]
        for c in range(C):
            S[m * 32 + c, c * 4 + l] = 1.0
    return S


def _np_TILE():
    T = np.zeros((C, C * SH), np.float32)
    for m in range(SH):
        for c in range(C):
            T[c, m * 32 + c] = 1.0
    return T


def _np_PERM():
    P = np.zeros((C * SH, C * SH), np.float32)
    for m in range(SH):
        for c in range(C):
            P[m * 32 + c, c * 16 + m] = 1.0
    return P


_S_CONST = _np_S()
_TILE_CONST = _np_TILE()
_PERM_CONST = _np_PERM()
_W3PERM = np.array([[c * 4 + l for c in range(C)] for l in range(4)],
                   np.int32).reshape(-1)


def _np_REP():
    M = np.zeros((C, 128), np.float32)
    for l in range(4):
        for c in range(C):
            M[c, l * 32 + c] = 1.0
    return M


def _np_SEL():
    M = np.zeros((128, C * SH), np.float32)
    for m in range(SH):
        for c in range(C):
            M[LOF[m] * 32 + c, m * 32 + c] = 1.0
    return M


def _np_EXPAND():
    M = np.zeros((SH, C * SH), np.float32)
    for m in range(SH):
        for c in range(C):
            M[m, m * 32 + c] = 1.0
    return M


_REP_CONST = _np_REP()
_SEL_CONST = _np_SEL().astype(np.float32)
_EXPAND_CONST = _np_EXPAND()


def _sph_harm_cols(x, y, z):
    s3 = np.sqrt(3.0); s15 = np.sqrt(15.0); s5 = np.sqrt(5.0)
    c1 = np.sqrt(35.0 / 8.0); c2 = np.sqrt(105.0); c3 = np.sqrt(21.0 / 8.0)
    c4 = np.sqrt(7.0) / 2.0; c5 = np.sqrt(105.0) / 2.0
    one = jnp.ones_like(x)
    return [
        one,
        s3 * x, s3 * y, s3 * z,
        s15 * x * y, s15 * y * z, (s5 / 2.0) * (3.0 * z * z - one),
        s15 * x * z, (s15 / 2.0) * (x * x - y * y),
        c1 * y * (3.0 * x * x - y * y), c2 * x * y * z,
        c3 * y * (5.0 * z * z - one), c4 * z * (5.0 * z * z - 3.0 * one),
        c3 * x * (5.0 * z * z - one), c5 * z * (x * x - y * y),
        c1 * x * (x * x - 3.0 * y * y),
    ]


def _edge_kernel(bases_ref, geom_ref, ids_ref, sj_ref, W1_ref, W2_ref,
                 W3p_ref, REP_ref, SEL_ref, EXP_ref, out_ref, acc_ref):
    b = pl.program_id(0)

    @pl.when(b == 0)
    def _():
        acc_ref[...] = jnp.zeros_like(acc_ref)

    @pl.when(b >= NBLK_E)
    def _():
        off = pl.multiple_of((b - NBLK_E) * NBN, NBN)
        out_ref[...] = acc_ref[pl.ds(off, NBN), :]

    @pl.when(b < NBLK_E)
    def _():
        _edge_block(bases_ref, geom_ref, ids_ref, sj_ref, W1_ref,
                    W2_ref, W3p_ref, REP_ref, SEL_ref, EXP_ref, acc_ref, b)


def _geom_kernel(pj_ref, pi_ref, out_ref):
    dT = jnp.transpose(pj_ref[:, :16] - pi_ref[:, :16])
    vx = dT[0:1, :]
    vy = dT[1:2, :]
    vz = dT[2:3, :]
    r2 = vx * vx + vy * vy + vz * vz
    r = jnp.sqrt(r2)
    rc = jnp.maximum(r, 1e-9)
    rinv = 1.0 / rc
    ux = vx * rinv; uy = vy * rinv; uz = vz * rinv
    YT = jnp.concatenate(_sph_harm_cols(ux, uy, uz), axis=0)

    p = 5.0
    ur = r * np.float32(1.0 / R_MAX)
    u5 = ur * ur * ur * ur * ur
    cut = (1.0 - 0.5 * (p + 1.0) * (p + 2.0) * u5 + p * (p + 2.0) * u5 * ur
           - 0.5 * p * (p + 1.0) * u5 * ur * ur)
    cut = cut * (ur < 1.0).astype(jnp.float32)
    scal = np.float32(np.sqrt(2.0 / R_MAX))
    amp = scal * rinv * cut
    theta = np.float32(np.pi / R_MAX) * r
    s1 = jnp.sin(theta)
    c2 = 2.0 * jnp.cos(theta)
    rows = [s1]
    prev2, prev1 = jnp.zeros_like(s1), s1
    for _ in range(NB - 1):
        cur = c2 * prev1 - prev2
        rows.append(cur)
        prev2, prev1 = prev1, cur
    efT = jnp.concatenate([rw * amp for rw in rows], axis=0)

    gT = jnp.concatenate([YT, efT], axis=0)
    out_ref[:, :24] = jnp.transpose(gT)
    out_ref[:, 24:] = jnp.zeros((EB, 104), jnp.float32)


def _geom_call(ppj, ppi):
    return pl.pallas_call(
        _geom_kernel,
        grid=(NBLK_E,),
        in_specs=[
            pl.BlockSpec((EB, 128), lambda b: (b, 0)),
            pl.BlockSpec((EB, 128), lambda b: (b, 0)),
        ],
        out_specs=pl.BlockSpec((EB, 128), lambda b: (b, 0)),
        out_shape=jax.ShapeDtypeStruct((E, 128), jnp.float32),
        compiler_params=pltpu.CompilerParams(
            dimension_semantics=("arbitrary",)),
        interpret=_INTERPRET,
    )(ppj, ppi)


def _edge_block(bases_ref, geom_ref, ids_ref, sj_ref, W1_ref, W2_ref,
                W3p_ref, REP_ref, SEL_ref, EXP_ref, acc_ref, b):
    def mm(a, bm, pt=jnp.float32):
        return jax.lax.dot_general(a, bm, (((1,), (0,)), ((), ())),
                                   preferred_element_type=pt)

    Y = geom_ref[:, :SH]
    ef = geom_ref[:, SH:SH + NB]

    h1 = jax.nn.silu(mm(ef, W1_ref[...]))
    h2 = jax.nn.silu(mm(h1, W2_ref[...]))
    R2 = mm(h2, W3p_ref[...])

    sj = sj_ref[:, :C]
    T = R2 * mm(sj, REP_ref[...])
    Tsel = mm(T.astype(jnp.bfloat16), SEL_ref[...]).astype(jnp.bfloat16)
    Yexp = mm(Y.astype(jnp.bfloat16), EXP_ref[...]).astype(jnp.bfloat16)
    msg = Tsel * Yexp

    ids = ids_ref[0]
    base = pl.multiple_of(bases_ref[b], 8)
    rel = jnp.broadcast_to(ids - base, (W, EB))
    iot = jax.lax.broadcasted_iota(jnp.int32, (W, EB), 0)
    onehotT = (rel == iot).astype(jnp.bfloat16)

    contrib = jax.lax.dot_general(
        onehotT, msg, (((1,), (0,)), ((), ())),
        preferred_element_type=jnp.float32)
    cur = acc_ref[pl.ds(base, W), :]
    acc_ref[pl.ds(base, W), :] = cur + contrib


def _edge_call(bases, geom, ids3, sj, W1, W2, W3p, REPc, SELc, EXPc):
    cap = lambda b: jnp.minimum(b, NBLK_E - 1)
    spec = pltpu.PrefetchScalarGridSpec(
        num_scalar_prefetch=1,
        grid=(NBLK_E + NBLK_N,),
        in_specs=[
            pl.BlockSpec((EB, 128), lambda b, s: (cap(b), 0)),
            pl.BlockSpec((1, 1, EB), lambda b, s: (cap(b), 0, 0)),
            pl.BlockSpec((EB, 128), lambda b, s: (cap(b), 0)),
            pl.BlockSpec((NB, 64), lambda b, s: (0, 0)),
            pl.BlockSpec((64, 64), lambda b, s: (0, 0)),
            pl.BlockSpec((64, 128), lambda b, s: (0, 0)),
            pl.BlockSpec((C, 128), lambda b, s: (0, 0)),
            pl.BlockSpec((128, C * SH), lambda b, s: (0, 0)),
            pl.BlockSpec((SH, C * SH), lambda b, s: (0, 0)),
        ],
        out_specs=pl.BlockSpec(
            (NBN, C * SH),
            lambda b, s: (jnp.maximum(b - NBLK_E, 0), 0)),
        scratch_shapes=[pltpu.VMEM((NPAD, C * SH), jnp.float32)],
    )
    return pl.pallas_call(
        _edge_kernel,
        grid_spec=spec,
        out_shape=jax.ShapeDtypeStruct((NPAD, C * SH), jnp.float32),
        compiler_params=pltpu.CompilerParams(
            dimension_semantics=("arbitrary",)),
        interpret=_INTERPRET,
    )(bases, geom, ids3, sj, W1, W2, W3p, REPc, SELc, EXPc)


def _node_kernel_first(x_ref, agg_ref, Wemb_ref, BM_ref, S_ref, Wp_ref,
                       T_ref, BO_ref, BS_ref, ve_ref, nf_ref):
    _node_body(x_ref, agg_ref, None, Wemb_ref, BM_ref, S_ref, Wp_ref, T_ref,
               BO_ref, BS_ref, ve_ref, nf_ref, first=True)


def _node_kernel_rest(x_ref, agg_ref, nfin_ref, BM_ref, S_ref, Wp_ref,
                      T_ref, BO_ref, BS_ref, ve_ref, nf_ref):
    _node_body(x_ref, agg_ref, nfin_ref, None, BM_ref, S_ref, Wp_ref, T_ref,
               BO_ref, BS_ref, ve_ref, nf_ref, first=False)


def _node_body(x_ref, agg_ref, nfin_ref, Wemb_ref, BM_ref, S_ref, Wp_ref,
               T_ref, BO_ref, BS_ref, ve_ref, nf_ref, first):
    x = x_ref[...]
    agg = agg_ref[...]

    def mm(a, b):
        return jax.lax.dot_general(a, b, (((1,), (0,)), ((), ())),
                                   preferred_element_type=jnp.float32)

    if first:
        h = mm(x, Wemb_ref[...])
        nf_cur = jnp.concatenate(
            [h, jnp.zeros((NBN, C * SH - C), jnp.float32)], axis=1)
    else:
        nf_cur = nfin_ref[...]

    mid = mm(agg, BM_ref[...])
    inv = mm(mid * mid, S_ref[...])
    g = jax.nn.silu(mm(inv, Wp_ref[...]))
    gt = mm(g, T_ref[...])
    ew = mm(x, ve_ref[...])
    sc = mm(nf_cur, BS_ref[...]) * ew
    nf_ref[...] = mm(mid, BO_ref[...]) * gt + sc


def _node_call(x, agg, nfin, Wemb, BM, S, Wp, T, BO, BS, ve, first):
    full = lambda shape: pl.BlockSpec(shape, lambda b: (0, 0))
    in_specs = [
        pl.BlockSpec((NBN, 16), lambda b: (b, 0)),
        pl.BlockSpec((NBN, C * SH), lambda b: (b, 0)),
    ]
    args = [x, agg]
    if first:
        kern = _node_kernel_first
        in_specs.append(full((16, C)))
        args.append(Wemb)
    else:
        kern = _node_kernel_rest
        in_specs.append(pl.BlockSpec((NBN, C * SH), lambda b: (b, 0)))
        args.append(nfin)
    in_specs += [full((C * SH, C * SH)), full((C * SH, C * 4)),
                 full((C * 4, C)), full((C, C * SH)),
                 full((C * SH, C * SH)), full((C * SH, C * SH)),
                 full((16, 1))]
    args += [BM, S, Wp, T, BO, BS, ve]
    return pl.pallas_call(
        kern,
        grid=(NBLK_N,),
        in_specs=in_specs,
        out_specs=pl.BlockSpec((NBN, C * SH), lambda b: (b, 0)),
        out_shape=jax.ShapeDtypeStruct((NPAD, C * SH), jnp.float32),
        compiler_params=pltpu.CompilerParams(
            dimension_semantics=("arbitrary",)),
        interpret=_INTERPRET,
    )(*args)


def _pool_kernel(nf_ref, bat_ref, PERM_ref, out_ref, acc, cnt):
    b = pl.program_id(0)

    @pl.when(b == 0)
    def _():
        acc[...] = jnp.zeros_like(acc)
        cnt[...] = jnp.zeros_like(cnt)

    bat = bat_ref[0]
    iot = jax.lax.broadcasted_iota(jnp.int32, (NG, NBN), 0)
    oh = (jnp.broadcast_to(bat, (NG, NBN)) == iot).astype(jnp.bfloat16)
    nfb = nf_ref[...].astype(jnp.bfloat16)

    def mm(a, b_, pt=jnp.float32):
        return jax.lax.dot_general(a, b_, (((1,), (0,)), ((), ())),
                                   preferred_element_type=pt)

    acc[...] = acc[...] + mm(oh, nfb)
    ones = jnp.ones((NBN, 128), jnp.bfloat16)
    cnt[...] = cnt[...] + mm(oh, ones)

    @pl.when(b == NBLK_N - 1)
    def _():
        c = jnp.maximum(cnt[:, 0:1], 1.0)
        out_ref[...] = mm(acc[...] / c, PERM_ref[...])


def _pool_call(nf, bat3, PERM):
    return pl.pallas_call(
        _pool_kernel,
        grid=(NBLK_N,),
        in_specs=[
            pl.BlockSpec((NBN, C * SH), lambda b: (b, 0)),
            pl.BlockSpec((1, 1, NBN), lambda b: (b, 0, 0)),
            pl.BlockSpec((C * SH, C * SH), lambda b: (0, 0)),
        ],
        out_specs=pl.BlockSpec((NG, C * SH), lambda b: (0, 0)),
        out_shape=jax.ShapeDtypeStruct((NG, C * SH), jnp.float32),
        scratch_shapes=[pltpu.VMEM((NG, C * SH), jnp.float32),
                        pltpu.VMEM((NG, 128), jnp.float32)],
        compiler_params=pltpu.CompilerParams(
            dimension_semantics=("arbitrary",)),
        interpret=_INTERPRET,
    )(nf, bat3, PERM)


def _sc_gather(table, idx, chunk):
    V, D = table.shape
    B = idx.shape[0]
    NW = 32
    b_per_w = B // NW
    nchunk = b_per_w // chunk
    mesh = plsc.VectorSubcoreMesh(core_axis_name="c", subcore_axis_name="s")

    @functools.partial(
        pl.kernel, mesh=mesh,
        out_type=jax.ShapeDtypeStruct((B, D), jnp.float32),
        compiler_params=pltpu.CompilerParams(use_tc_tiling_on_sc=True),
        scratch_types=[
            pltpu.VMEM((chunk,), jnp.int32),
            pltpu.VMEM((chunk, D), jnp.float32),
            pltpu.SemaphoreType.DMA,
        ],
    )
    def k(table_hbm, idx_hbm, out_hbm, idx_v, rows_v, sem):
        wid = lax.axis_index("s") * 2 + lax.axis_index("c")
        for ci in range(nchunk):
            base = wid * b_per_w + ci * chunk
            pltpu.sync_copy(idx_hbm.at[pl.ds(base, chunk)], idx_v)
            pltpu.async_copy(table_hbm.at[idx_v], rows_v, sem).wait()
            pltpu.sync_copy(rows_v, out_hbm.at[pl.ds(base, chunk)])

    return k(table, idx)


def _embed_kernel(x_ref, W_ref, h_ref):
    h_ref[...] = jax.lax.dot_general(
        x_ref[...], W_ref[...], (((1,), (0,)), ((), ())),
        preferred_element_type=jnp.float32)


def _embed_call(x_pad, Wemb128):
    return pl.pallas_call(
        _embed_kernel,
        out_shape=jax.ShapeDtypeStruct((NPAD, 128), jnp.float32),
        interpret=_INTERPRET,
    )(x_pad, Wemb128)


def _big_blockdiag(w4, scale=1.0):
    z = jnp.zeros((C, C), jnp.float32)
    return jnp.concatenate(
        [jnp.concatenate([z] * m + [w4[LOF[m]] * scale]
                         + [z] * (SH - 1 - m), axis=1)
         for m in range(SH)], axis=0)


def kernel(x, pos, batch, edge_index, W_embed, W1_0, W2_0, W3_0, mix_0,
           Wsc_0, velem_0, Wprod_0, Wout_0, W1_1, W2_1, W3_1, mix_1, Wsc_1,
           velem_1, Wprod_1, Wout_1):
    f32 = jnp.float32
    x = x.astype(f32)
    j = edge_index[0].astype(jnp.int32)
    i = edge_index[1].astype(jnp.int32)
    bat = batch.astype(jnp.int32)

    order = jnp.argsort(i)
    i_s = i[order]
    j_s = j[order]
    bases = ((i_s[::EB] // 8) * 8).astype(jnp.int32)
    ids3 = i_s.reshape(NBLK_E, 1, EB)

    ppos = jnp.pad(pos.astype(f32), ((0, 0), (0, 125)))
    ppj = _sc_gather(ppos, j_s, 200)
    ppi = _sc_gather(ppos, i_s, 200)
    geom = _geom_call(ppj, ppi)
    REPc = jnp.asarray(_REP_CONST)
    SELc = jnp.asarray(_SEL_CONST).astype(jnp.bfloat16)
    EXPc = jnp.asarray(_EXPAND_CONST).astype(jnp.bfloat16)

    x_pad = jnp.pad(x, ((0, NPAD - N), (0, 16 - NE)))
    Wemb_pad = jnp.pad(W_embed.astype(f32), ((0, 16 - NE), (0, 0)))

    S = jnp.asarray(_S_CONST)
    T = jnp.asarray(_TILE_CONST)
    PERM = jnp.asarray(_PERM_CONST)

    Wemb128 = jnp.pad(Wemb_pad, ((0, 0), (0, 128 - C)))
    h = _embed_call(x_pad, Wemb128)
    bat3 = jnp.pad(bat, (0, NPAD - N), constant_values=NG).reshape(
        NBLK_N, 1, NBN)

    layer_params = [
        (W1_0, W2_0, W3_0, mix_0, Wsc_0, velem_0, Wprod_0, Wout_0),
        (W1_1, W2_1, W3_1, mix_1, Wsc_1, velem_1, Wprod_1, Wout_1),
    ]

    nf = None
    s_table = h[:N]
    for li, (W1, W2, W3, mix, Wsc, velem, Wprod, Wout) in enumerate(
            layer_params):
        W3p = W3.astype(f32)[:, _W3PERM]
        BM = _big_blockdiag(mix.astype(f32), scale=1.0 / AVG_NEIGH)
        BS = _big_blockdiag(Wsc.astype(f32))
        BO = _big_blockdiag(Wout.astype(f32))
        ve = jnp.pad(velem.astype(f32), (0, 16 - NE)).reshape(16, 1)

        sj = _sc_gather(s_table, j_s, 200)
        agg = _edge_call(bases, geom, ids3, sj, W1.astype(f32),
                         W2.astype(f32), W3p, REPc, SELc, EXPc)
        nf = _node_call(x_pad, agg, nf, Wemb_pad, BM, S,
                        Wprod.astype(f32), T, BO, BS, ve, first=(li == 0))
        s_table = jnp.pad(nf[:N, :C], ((0, 0), (0, 128 - C)))

    return _pool_call(nf, bat3, PERM)


# speedup vs baseline: 3.4524x; 1.0000x over previous
import functools

import jax
import jax.numpy as jnp
import numpy as np
from jax import lax
from jax.experimental import pallas as pl
from jax.experimental.pallas import tpu as pltpu
from jax.experimental.pallas import tpu_sc as plsc

N = 10000
E = 160000
NE = 10
C = 32
SH = 16
NB = 8
NG = 64
R_MAX = 5.0
AVG_NEIGH = 10.0
L_SLICES = [(0, 1), (1, 4), (4, 9), (9, 16)]
LOF = [0, 1, 1, 1, 2, 2, 2, 2, 2, 3, 3, 3, 3, 3, 3, 3]

EB = 3200
NBLK_E = E // EB
W = 256
NPAD = 11264
NBN = 1024
NBLK_N = NPAD // NBN

_INTERPRET = False


def _np_S():
    S = np.zeros((C * SH, C * 4), np.float32)
    for m in range(SH):
        l = LOF[m]
        for c in range(C):
            S[m * 32 + c, c * 4 + l] = 1.0
    return S


def _np_TILE():
    T = np.zeros((C, C * SH), np.float32)
    for m in range(SH):
        for c in range(C):
            T[c, m * 32 + c] = 1.0
    return T


def _np_PERM():
    P = np.zeros((C * SH, C * SH), np.float32)
    for m in range(SH):
        for c in range(C):
            P[m * 32 + c, c * 16 + m] = 1.0
    return P


_S_CONST = _np_S()
_TILE_CONST = _np_TILE()
_PERM_CONST = _np_PERM()
_W3PERM = np.array([[c * 4 + l for c in range(C)] for l in range(4)],
                   np.int32).reshape(-1)


def _np_REP():
    M = np.zeros((C, 128), np.float32)
    for l in range(4):
        for c in range(C):
            M[c, l * 32 + c] = 1.0
    return M


def _np_SEL():
    M = np.zeros((128, C * SH), np.float32)
    for m in range(SH):
        for c in range(C):
            M[LOF[m] * 32 + c, m * 32 + c] = 1.0
    return M


def _np_EXPAND():
    M = np.zeros((SH, C * SH), np.float32)
    for m in range(SH):
        for c in range(C):
            M[m, m * 32 + c] = 1.0
    return M


_REP_CONST = _np_REP()
_SEL_CONST = _np_SEL().astype(np.float32)
_EXPAND_CONST = _np_EXPAND()


def _sph_harm_cols(x, y, z):
    s3 = np.sqrt(3.0); s15 = np.sqrt(15.0); s5 = np.sqrt(5.0)
    c1 = np.sqrt(35.0 / 8.0); c2 = np.sqrt(105.0); c3 = np.sqrt(21.0 / 8.0)
    c4 = np.sqrt(7.0) / 2.0; c5 = np.sqrt(105.0) / 2.0
    one = jnp.ones_like(x)
    return [
        one,
        s3 * x, s3 * y, s3 * z,
        s15 * x * y, s15 * y * z, (s5 / 2.0) * (3.0 * z * z - one),
        s15 * x * z, (s15 / 2.0) * (x * x - y * y),
        c1 * y * (3.0 * x * x - y * y), c2 * x * y * z,
        c3 * y * (5.0 * z * z - one), c4 * z * (5.0 * z * z - 3.0 * one),
        c3 * x * (5.0 * z * z - one), c5 * z * (x * x - y * y),
        c1 * x * (x * x - 3.0 * y * y),
    ]


def _edge_kernel(bases_ref, geom_ref, ids_ref, sj_ref, W1_ref, W2_ref,
                 W3p_ref, REP_ref, SEL_ref, EXP_ref, out_ref, acc_ref):
    b = pl.program_id(0)

    @pl.when(b == 0)
    def _():
        acc_ref[...] = jnp.zeros_like(acc_ref)

    @pl.when(b >= NBLK_E)
    def _():
        off = pl.multiple_of((b - NBLK_E) * NBN, NBN)
        out_ref[...] = acc_ref[pl.ds(off, NBN), :]

    @pl.when(b < NBLK_E)
    def _():
        _edge_block(bases_ref, geom_ref, ids_ref, sj_ref, W1_ref,
                    W2_ref, W3p_ref, REP_ref, SEL_ref, EXP_ref, acc_ref, b)


def _geom_kernel(pj_ref, pi_ref, out_ref):
    dT = jnp.transpose(pj_ref[:, :16] - pi_ref[:, :16])
    vx = dT[0:1, :]
    vy = dT[1:2, :]
    vz = dT[2:3, :]
    r2 = vx * vx + vy * vy + vz * vz
    r = jnp.sqrt(r2)
    rc = jnp.maximum(r, 1e-9)
    rinv = 1.0 / rc
    ux = vx * rinv; uy = vy * rinv; uz = vz * rinv
    YT = jnp.concatenate(_sph_harm_cols(ux, uy, uz), axis=0)

    p = 5.0
    ur = r * np.float32(1.0 / R_MAX)
    u5 = ur * ur * ur * ur * ur
    cut = (1.0 - 0.5 * (p + 1.0) * (p + 2.0) * u5 + p * (p + 2.0) * u5 * ur
           - 0.5 * p * (p + 1.0) * u5 * ur * ur)
    cut = cut * (ur < 1.0).astype(jnp.float32)
    scal = np.float32(np.sqrt(2.0 / R_MAX))
    amp = scal * rinv * cut
    theta = np.float32(np.pi / R_MAX) * r
    s1 = jnp.sin(theta)
    c2 = 2.0 * jnp.cos(theta)
    rows = [s1]
    prev2, prev1 = jnp.zeros_like(s1), s1
    for _ in range(NB - 1):
        cur = c2 * prev1 - prev2
        rows.append(cur)
        prev2, prev1 = prev1, cur
    efT = jnp.concatenate([rw * amp for rw in rows], axis=0)

    gT = jnp.concatenate([YT, efT], axis=0)
    out_ref[:, :24] = jnp.transpose(gT)
    out_ref[:, 24:] = jnp.zeros((EB, 104), jnp.float32)


def _geom_call(ppj, ppi):
    return pl.pallas_call(
        _geom_kernel,
        grid=(NBLK_E,),
        in_specs=[
            pl.BlockSpec((EB, 128), lambda b: (b, 0)),
            pl.BlockSpec((EB, 128), lambda b: (b, 0)),
        ],
        out_specs=pl.BlockSpec((EB, 128), lambda b: (b, 0)),
        out_shape=jax.ShapeDtypeStruct((E, 128), jnp.float32),
        compiler_params=pltpu.CompilerParams(
            dimension_semantics=("arbitrary",)),
        interpret=_INTERPRET,
    )(ppj, ppi)


def _edge_block(bases_ref, geom_ref, ids_ref, sj_ref, W1_ref, W2_ref,
                W3p_ref, REP_ref, SEL_ref, EXP_ref, acc_ref, b):
    def mm(a, bm, pt=jnp.float32):
        return jax.lax.dot_general(a, bm, (((1,), (0,)), ((), ())),
                                   preferred_element_type=pt)

    Y = geom_ref[:, :SH]
    ef = geom_ref[:, SH:SH + NB]

    h1 = jax.nn.silu(mm(ef, W1_ref[...]))
    h2 = jax.nn.silu(mm(h1, W2_ref[...]))
    R2 = mm(h2.astype(jnp.bfloat16), W3p_ref[...])

    sj = sj_ref[:, :C]
    T = R2 * mm(sj, REP_ref[...])
    Tsel = mm(T.astype(jnp.bfloat16), SEL_ref[...]).astype(jnp.bfloat16)
    Yexp = mm(Y.astype(jnp.bfloat16), EXP_ref[...]).astype(jnp.bfloat16)
    msg = Tsel * Yexp

    ids = ids_ref[0]
    base = pl.multiple_of(bases_ref[b], 8)
    rel = jnp.broadcast_to(ids - base, (W, EB))
    iot = jax.lax.broadcasted_iota(jnp.int32, (W, EB), 0)
    onehotT = (rel == iot).astype(jnp.bfloat16)

    contrib = jax.lax.dot_general(
        onehotT, msg, (((1,), (0,)), ((), ())),
        preferred_element_type=jnp.float32)
    cur = acc_ref[pl.ds(base, W), :]
    acc_ref[pl.ds(base, W), :] = cur + contrib


def _edge_call(bases, geom, ids3, sj, W1, W2, W3p, REPc, SELc, EXPc):
    cap = lambda b: jnp.minimum(b, NBLK_E - 1)
    spec = pltpu.PrefetchScalarGridSpec(
        num_scalar_prefetch=1,
        grid=(NBLK_E + NBLK_N,),
        in_specs=[
            pl.BlockSpec((EB, 128), lambda b, s: (cap(b), 0)),
            pl.BlockSpec((1, 1, EB), lambda b, s: (cap(b), 0, 0)),
            pl.BlockSpec((EB, 128), lambda b, s: (cap(b), 0)),
            pl.BlockSpec((NB, 64), lambda b, s: (0, 0)),
            pl.BlockSpec((64, 64), lambda b, s: (0, 0)),
            pl.BlockSpec((64, 128), lambda b, s: (0, 0)),
            pl.BlockSpec((C, 128), lambda b, s: (0, 0)),
            pl.BlockSpec((128, C * SH), lambda b, s: (0, 0)),
            pl.BlockSpec((SH, C * SH), lambda b, s: (0, 0)),
        ],
        out_specs=pl.BlockSpec(
            (NBN, C * SH),
            lambda b, s: (jnp.maximum(b - NBLK_E, 0), 0)),
        scratch_shapes=[pltpu.VMEM((NPAD, C * SH), jnp.float32)],
    )
    return pl.pallas_call(
        _edge_kernel,
        grid_spec=spec,
        out_shape=jax.ShapeDtypeStruct((NPAD, C * SH), jnp.float32),
        compiler_params=pltpu.CompilerParams(
            dimension_semantics=("arbitrary",)),
        interpret=_INTERPRET,
    )(bases, geom, ids3, sj, W1, W2, W3p, REPc, SELc, EXPc)


def _node_kernel_first(x_ref, agg_ref, Wemb_ref, BM_ref, S_ref, Wp_ref,
                       T_ref, BO_ref, BS_ref, ve_ref, nf_ref):
    _node_body(x_ref, agg_ref, None, Wemb_ref, BM_ref, S_ref, Wp_ref, T_ref,
               BO_ref, BS_ref, ve_ref, nf_ref, first=True)


def _node_kernel_rest(x_ref, agg_ref, nfin_ref, BM_ref, S_ref, Wp_ref,
                      T_ref, BO_ref, BS_ref, ve_ref, nf_ref):
    _node_body(x_ref, agg_ref, nfin_ref, None, BM_ref, S_ref, Wp_ref, T_ref,
               BO_ref, BS_ref, ve_ref, nf_ref, first=False)


def _node_body(x_ref, agg_ref, nfin_ref, Wemb_ref, BM_ref, S_ref, Wp_ref,
               T_ref, BO_ref, BS_ref, ve_ref, nf_ref, first):
    x = x_ref[...]
    agg = agg_ref[...]

    def mm(a, b):
        return jax.lax.dot_general(a, b, (((1,), (0,)), ((), ())),
                                   preferred_element_type=jnp.float32)

    if first:
        h = mm(x, Wemb_ref[...])
        nf_cur = jnp.concatenate(
            [h, jnp.zeros((NBN, C * SH - C), jnp.float32)], axis=1)
    else:
        nf_cur = nfin_ref[...]

    bf = jnp.bfloat16
    mid = mm(agg.astype(bf), BM_ref[...])
    inv = mm(mid * mid, S_ref[...])
    g = jax.nn.silu(mm(inv, Wp_ref[...]))
    gt = mm(g, T_ref[...])
    ew = mm(x, ve_ref[...])
    sc = mm(nf_cur.astype(bf), BS_ref[...]) * ew
    nf_ref[...] = mm(mid.astype(bf), BO_ref[...]) * gt + sc


def _node_call(x, agg, nfin, Wemb, BM, S, Wp, T, BO, BS, ve, first):
    full = lambda shape: pl.BlockSpec(shape, lambda b: (0, 0))
    in_specs = [
        pl.BlockSpec((NBN, 16), lambda b: (b, 0)),
        pl.BlockSpec((NBN, C * SH), lambda b: (b, 0)),
    ]
    args = [x, agg]
    if first:
        kern = _node_kernel_first
        in_specs.append(full((16, C)))
        args.append(Wemb)
    else:
        kern = _node_kernel_rest
        in_specs.append(pl.BlockSpec((NBN, C * SH), lambda b: (b, 0)))
        args.append(nfin)
    in_specs += [full((C * SH, C * SH)), full((C * SH, C * 4)),
                 full((C * 4, C)), full((C, C * SH)),
                 full((C * SH, C * SH)), full((C * SH, C * SH)),
                 full((16, 1))]
    args += [BM, S, Wp, T, BO, BS, ve]
    return pl.pallas_call(
        kern,
        grid=(NBLK_N,),
        in_specs=in_specs,
        out_specs=pl.BlockSpec((NBN, C * SH), lambda b: (b, 0)),
        out_shape=jax.ShapeDtypeStruct((NPAD, C * SH), jnp.float32),
        compiler_params=pltpu.CompilerParams(
            dimension_semantics=("arbitrary",)),
        interpret=_INTERPRET,
    )(*args)


def _pool_kernel(nf_ref, bat_ref, PERM_ref, out_ref, acc, cnt):
    b = pl.program_id(0)

    @pl.when(b == 0)
    def _():
        acc[...] = jnp.zeros_like(acc)
        cnt[...] = jnp.zeros_like(cnt)

    bat = bat_ref[0]
    iot = jax.lax.broadcasted_iota(jnp.int32, (NG, NBN), 0)
    oh = (jnp.broadcast_to(bat, (NG, NBN)) == iot).astype(jnp.bfloat16)
    nfb = nf_ref[...].astype(jnp.bfloat16)

    def mm(a, b_, pt=jnp.float32):
        return jax.lax.dot_general(a, b_, (((1,), (0,)), ((), ())),
                                   preferred_element_type=pt)

    acc[...] = acc[...] + mm(oh, nfb)
    ones = jnp.ones((NBN, 128), jnp.bfloat16)
    cnt[...] = cnt[...] + mm(oh, ones)

    @pl.when(b == NBLK_N - 1)
    def _():
        c = jnp.maximum(cnt[:, 0:1], 1.0)
        out_ref[...] = mm(acc[...] / c, PERM_ref[...])


def _pool_call(nf, bat3, PERM):
    return pl.pallas_call(
        _pool_kernel,
        grid=(NBLK_N,),
        in_specs=[
            pl.BlockSpec((NBN, C * SH), lambda b: (b, 0)),
            pl.BlockSpec((1, 1, NBN), lambda b: (b, 0, 0)),
            pl.BlockSpec((C * SH, C * SH), lambda b: (0, 0)),
        ],
        out_specs=pl.BlockSpec((NG, C * SH), lambda b: (0, 0)),
        out_shape=jax.ShapeDtypeStruct((NG, C * SH), jnp.float32),
        scratch_shapes=[pltpu.VMEM((NG, C * SH), jnp.float32),
                        pltpu.VMEM((NG, 128), jnp.float32)],
        compiler_params=pltpu.CompilerParams(
            dimension_semantics=("arbitrary",)),
        interpret=_INTERPRET,
    )(nf, bat3, PERM)


def _sc_gather(table, idx, chunk):
    V, D = table.shape
    B = idx.shape[0]
    NW = 32
    b_per_w = B // NW
    nchunk = b_per_w // chunk
    mesh = plsc.VectorSubcoreMesh(core_axis_name="c", subcore_axis_name="s")

    @functools.partial(
        pl.kernel, mesh=mesh,
        out_type=jax.ShapeDtypeStruct((B, D), jnp.float32),
        compiler_params=pltpu.CompilerParams(use_tc_tiling_on_sc=True),
        scratch_types=[
            pltpu.VMEM((chunk,), jnp.int32),
            pltpu.VMEM((chunk, D), jnp.float32),
            pltpu.SemaphoreType.DMA,
        ],
    )
    def k(table_hbm, idx_hbm, out_hbm, idx_v, rows_v, sem):
        wid = lax.axis_index("s") * 2 + lax.axis_index("c")
        for ci in range(nchunk):
            base = wid * b_per_w + ci * chunk
            pltpu.sync_copy(idx_hbm.at[pl.ds(base, chunk)], idx_v)
            pltpu.async_copy(table_hbm.at[idx_v], rows_v, sem).wait()
            pltpu.sync_copy(rows_v, out_hbm.at[pl.ds(base, chunk)])

    return k(table, idx)


def _embed_kernel(x_ref, W_ref, h_ref):
    h_ref[...] = jax.lax.dot_general(
        x_ref[...], W_ref[...], (((1,), (0,)), ((), ())),
        preferred_element_type=jnp.float32)


def _embed_call(x_pad, Wemb128):
    return pl.pallas_call(
        _embed_kernel,
        out_shape=jax.ShapeDtypeStruct((NPAD, 128), jnp.float32),
        interpret=_INTERPRET,
    )(x_pad, Wemb128)


def _big_blockdiag(w4, scale=1.0):
    z = jnp.zeros((C, C), jnp.float32)
    return jnp.concatenate(
        [jnp.concatenate([z] * m + [w4[LOF[m]] * scale]
                         + [z] * (SH - 1 - m), axis=1)
         for m in range(SH)], axis=0)


def kernel(x, pos, batch, edge_index, W_embed, W1_0, W2_0, W3_0, mix_0,
           Wsc_0, velem_0, Wprod_0, Wout_0, W1_1, W2_1, W3_1, mix_1, Wsc_1,
           velem_1, Wprod_1, Wout_1):
    f32 = jnp.float32
    x = x.astype(f32)
    j = edge_index[0].astype(jnp.int32)
    i = edge_index[1].astype(jnp.int32)
    bat = batch.astype(jnp.int32)

    order = jnp.argsort(i)
    i_s = i[order]
    j_s = j[order]
    bases = ((i_s[::EB] // 8) * 8).astype(jnp.int32)
    ids3 = i_s.reshape(NBLK_E, 1, EB)

    ppos = jnp.pad(pos.astype(f32), ((0, 0), (0, 125)))
    ppj = _sc_gather(ppos, j_s, 200)
    ppi = _sc_gather(ppos, i_s, 200)
    geom = _geom_call(ppj, ppi)
    REPc = jnp.asarray(_REP_CONST)
    SELc = jnp.asarray(_SEL_CONST).astype(jnp.bfloat16)
    EXPc = jnp.asarray(_EXPAND_CONST).astype(jnp.bfloat16)

    x_pad = jnp.pad(x, ((0, NPAD - N), (0, 16 - NE)))
    Wemb_pad = jnp.pad(W_embed.astype(f32), ((0, 16 - NE), (0, 0)))

    S = jnp.asarray(_S_CONST)
    T = jnp.asarray(_TILE_CONST)
    PERM = jnp.asarray(_PERM_CONST)

    Wemb128 = jnp.pad(Wemb_pad, ((0, 0), (0, 128 - C)))
    h = _embed_call(x_pad, Wemb128)
    bat3 = jnp.pad(bat, (0, NPAD - N), constant_values=NG).reshape(
        NBLK_N, 1, NBN)

    layer_params = [
        (W1_0, W2_0, W3_0, mix_0, Wsc_0, velem_0, Wprod_0, Wout_0),
        (W1_1, W2_1, W3_1, mix_1, Wsc_1, velem_1, Wprod_1, Wout_1),
    ]

    nf = None
    s_table = h[:N]
    for li, (W1, W2, W3, mix, Wsc, velem, Wprod, Wout) in enumerate(
            layer_params):
        W3p = W3.astype(f32)[:, _W3PERM].astype(jnp.bfloat16)
        BM = _big_blockdiag(mix.astype(f32),
                            scale=1.0 / AVG_NEIGH).astype(jnp.bfloat16)
        BS = _big_blockdiag(Wsc.astype(f32)).astype(jnp.bfloat16)
        BO = _big_blockdiag(Wout.astype(f32)).astype(jnp.bfloat16)
        ve = jnp.pad(velem.astype(f32), (0, 16 - NE)).reshape(16, 1)

        sj = _sc_gather(s_table, j_s, 200)
        agg = _edge_call(bases, geom, ids3, sj, W1.astype(f32),
                         W2.astype(f32), W3p, REPc, SELc, EXPc)
        nf = _node_call(x_pad, agg, nf, Wemb_pad, BM, S,
                        Wprod.astype(f32), T, BO, BS, ve, first=(li == 0))
        s_table = jnp.pad(nf[:N, :C], ((0, 0), (0, 128 - C)))

    return _pool_call(nf, bat3, PERM)


# speedup vs baseline: 3.7371x; 1.0824x over previous
import functools

import jax
import jax.numpy as jnp
import numpy as np
from jax import lax
from jax.experimental import pallas as pl
from jax.experimental.pallas import tpu as pltpu
from jax.experimental.pallas import tpu_sc as plsc

N = 10000
E = 160000
NE = 10
C = 32
SH = 16
NB = 8
NG = 64
R_MAX = 5.0
AVG_NEIGH = 10.0
L_SLICES = [(0, 1), (1, 4), (4, 9), (9, 16)]
LOF = [0, 1, 1, 1, 2, 2, 2, 2, 2, 3, 3, 3, 3, 3, 3, 3]

EB = 3200
NBLK_E = E // EB
W = 256
NPAD = 11264
NBN = 1024
NBLK_N = NPAD // NBN

_INTERPRET = False


def _np_S():
    S = np.zeros((C * SH, C * 4), np.float32)
    for m in range(SH):
        l = LOF[m]
        for c in range(C):
            S[m * 32 + c, c * 4 + l] = 1.0
    return S


def _np_TILE():
    T = np.zeros((C, C * SH), np.float32)
    for m in range(SH):
        for c in range(C):
            T[c, m * 32 + c] = 1.0
    return T


def _np_PERM():
    P = np.zeros((C * SH, C * SH), np.float32)
    for m in range(SH):
        for c in range(C):
            P[m * 32 + c, c * 16 + m] = 1.0
    return P


_S_CONST = _np_S()
_TILE_CONST = _np_TILE()
_PERM_CONST = _np_PERM()
_W3PERM = np.array([[c * 4 + l for c in range(C)] for l in range(4)],
                   np.int32).reshape(-1)


def _np_REP():
    M = np.zeros((C, 128), np.float32)
    for l in range(4):
        for c in range(C):
            M[c, l * 32 + c] = 1.0
    return M


def _np_SEL():
    M = np.zeros((128, C * SH), np.float32)
    for m in range(SH):
        for c in range(C):
            M[LOF[m] * 32 + c, m * 32 + c] = 1.0
    return M


def _np_EXPAND():
    M = np.zeros((SH, C * SH), np.float32)
    for m in range(SH):
        for c in range(C):
            M[m, m * 32 + c] = 1.0
    return M


_REP_CONST = _np_REP()
_SEL_CONST = _np_SEL().astype(np.float32)
_EXPAND_CONST = _np_EXPAND()


def _sph_harm_cols(x, y, z):
    s3 = np.sqrt(3.0); s15 = np.sqrt(15.0); s5 = np.sqrt(5.0)
    c1 = np.sqrt(35.0 / 8.0); c2 = np.sqrt(105.0); c3 = np.sqrt(21.0 / 8.0)
    c4 = np.sqrt(7.0) / 2.0; c5 = np.sqrt(105.0) / 2.0
    one = jnp.ones_like(x)
    return [
        one,
        s3 * x, s3 * y, s3 * z,
        s15 * x * y, s15 * y * z, (s5 / 2.0) * (3.0 * z * z - one),
        s15 * x * z, (s15 / 2.0) * (x * x - y * y),
        c1 * y * (3.0 * x * x - y * y), c2 * x * y * z,
        c3 * y * (5.0 * z * z - one), c4 * z * (5.0 * z * z - 3.0 * one),
        c3 * x * (5.0 * z * z - one), c5 * z * (x * x - y * y),
        c1 * x * (x * x - 3.0 * y * y),
    ]


def _edge_kernel(bases_ref, geom_ref, ids_ref, sj_ref, W1_ref, W2_ref,
                 W3p_ref, REP_ref, SEL_ref, EXP_ref, out_ref, acc_ref):
    b = pl.program_id(0)

    @pl.when(b == 0)
    def _():
        acc_ref[...] = jnp.zeros_like(acc_ref)

    @pl.when(b >= NBLK_E)
    def _():
        off = pl.multiple_of((b - NBLK_E) * NBN, NBN)
        out_ref[...] = acc_ref[pl.ds(off, NBN), :]

    @pl.when(b < NBLK_E)
    def _():
        _edge_block(bases_ref, geom_ref, ids_ref, sj_ref, W1_ref,
                    W2_ref, W3p_ref, REP_ref, SEL_ref, EXP_ref, acc_ref, b)


def _geom_kernel(pj_ref, pi_ref, out_ref):
    dT = jnp.transpose(pj_ref[:, :16] - pi_ref[:, :16])
    vx = dT[0:1, :]
    vy = dT[1:2, :]
    vz = dT[2:3, :]
    r2 = vx * vx + vy * vy + vz * vz
    r = jnp.sqrt(r2)
    rc = jnp.maximum(r, 1e-9)
    rinv = 1.0 / rc
    ux = vx * rinv; uy = vy * rinv; uz = vz * rinv
    YT = jnp.concatenate(_sph_harm_cols(ux, uy, uz), axis=0)

    p = 5.0
    ur = r * np.float32(1.0 / R_MAX)
    u5 = ur * ur * ur * ur * ur
    cut = (1.0 - 0.5 * (p + 1.0) * (p + 2.0) * u5 + p * (p + 2.0) * u5 * ur
           - 0.5 * p * (p + 1.0) * u5 * ur * ur)
    cut = cut * (ur < 1.0).astype(jnp.float32)
    scal = np.float32(np.sqrt(2.0 / R_MAX))
    amp = scal * rinv * cut
    theta = np.float32(np.pi / R_MAX) * r
    s1 = jnp.sin(theta)
    c2 = 2.0 * jnp.cos(theta)
    rows = [s1]
    prev2, prev1 = jnp.zeros_like(s1), s1
    for _ in range(NB - 1):
        cur = c2 * prev1 - prev2
        rows.append(cur)
        prev2, prev1 = prev1, cur
    efT = jnp.concatenate([rw * amp for rw in rows], axis=0)

    gT = jnp.concatenate([YT, efT], axis=0)
    out_ref[:, :24] = jnp.transpose(gT)
    out_ref[:, 24:] = jnp.zeros((EB, 104), jnp.float32)


def _geom_call(ppj, ppi):
    return pl.pallas_call(
        _geom_kernel,
        grid=(NBLK_E,),
        in_specs=[
            pl.BlockSpec((EB, 128), lambda b: (b, 0)),
            pl.BlockSpec((EB, 128), lambda b: (b, 0)),
        ],
        out_specs=pl.BlockSpec((EB, 128), lambda b: (b, 0)),
        out_shape=jax.ShapeDtypeStruct((E, 128), jnp.float32),
        compiler_params=pltpu.CompilerParams(
            dimension_semantics=("arbitrary",)),
        interpret=_INTERPRET,
    )(ppj, ppi)


def _edge_block(bases_ref, geom_ref, ids_ref, sj_ref, W1_ref, W2_ref,
                W3p_ref, REP_ref, SEL_ref, EXP_ref, acc_ref, b):
    def mm(a, bm, pt=jnp.float32):
        return jax.lax.dot_general(a, bm, (((1,), (0,)), ((), ())),
                                   preferred_element_type=pt)

    Y = geom_ref[:, :SH]
    ef = geom_ref[:, SH:SH + NB]

    h1 = jax.nn.silu(mm(ef, W1_ref[...]))
    h2 = jax.nn.silu(mm(h1, W2_ref[...]))
    R2 = mm(h2.astype(jnp.bfloat16), W3p_ref[...])

    sj = sj_ref[:, :C]
    T = R2 * mm(sj, REP_ref[...])
    Tsel = mm(T.astype(jnp.bfloat16), SEL_ref[...]).astype(jnp.bfloat16)
    Yexp = mm(Y.astype(jnp.bfloat16), EXP_ref[...]).astype(jnp.bfloat16)
    msg = Tsel * Yexp

    ids = ids_ref[0]
    base = pl.multiple_of(bases_ref[b], 8)
    rel = jnp.broadcast_to(ids - base, (W, EB))
    iot = jax.lax.broadcasted_iota(jnp.int32, (W, EB), 0)
    onehotT = (rel == iot).astype(jnp.bfloat16)

    contrib = jax.lax.dot_general(
        onehotT, msg, (((1,), (0,)), ((), ())),
        preferred_element_type=jnp.float32)
    cur = acc_ref[pl.ds(base, W), :]
    acc_ref[pl.ds(base, W), :] = cur + contrib


def _edge_call(bases, geom, ids3, sj, W1, W2, W3p, REPc, SELc, EXPc):
    cap = lambda b: jnp.minimum(b, NBLK_E - 1)
    spec = pltpu.PrefetchScalarGridSpec(
        num_scalar_prefetch=1,
        grid=(NBLK_E + NBLK_N,),
        in_specs=[
            pl.BlockSpec((EB, 128), lambda b, s: (cap(b), 0)),
            pl.BlockSpec((1, 1, EB), lambda b, s: (cap(b), 0, 0)),
            pl.BlockSpec((EB, 128), lambda b, s: (cap(b), 0)),
            pl.BlockSpec((NB, 64), lambda b, s: (0, 0)),
            pl.BlockSpec((64, 64), lambda b, s: (0, 0)),
            pl.BlockSpec((64, 128), lambda b, s: (0, 0)),
            pl.BlockSpec((C, 128), lambda b, s: (0, 0)),
            pl.BlockSpec((128, C * SH), lambda b, s: (0, 0)),
            pl.BlockSpec((SH, C * SH), lambda b, s: (0, 0)),
        ],
        out_specs=pl.BlockSpec(
            (NBN, C * SH),
            lambda b, s: (jnp.maximum(b - NBLK_E, 0), 0)),
        scratch_shapes=[pltpu.VMEM((NPAD, C * SH), jnp.float32)],
    )
    return pl.pallas_call(
        _edge_kernel,
        grid_spec=spec,
        out_shape=jax.ShapeDtypeStruct((NPAD, C * SH), jnp.float32),
        compiler_params=pltpu.CompilerParams(
            dimension_semantics=("arbitrary",)),
        interpret=_INTERPRET,
    )(bases, geom, ids3, sj, W1, W2, W3p, REPc, SELc, EXPc)


def _node_kernel_first(x_ref, agg_ref, Wemb_ref, BM_ref, S_ref, Wp_ref,
                       T_ref, BO_ref, BS_ref, ve_ref, nf_ref):
    _node_body(x_ref, agg_ref, None, Wemb_ref, BM_ref, S_ref, Wp_ref, T_ref,
               BO_ref, BS_ref, ve_ref, nf_ref, first=True)


def _node_kernel_rest(x_ref, agg_ref, nfin_ref, BM_ref, S_ref, Wp_ref,
                      T_ref, BO_ref, BS_ref, ve_ref, nf_ref):
    _node_body(x_ref, agg_ref, nfin_ref, None, BM_ref, S_ref, Wp_ref, T_ref,
               BO_ref, BS_ref, ve_ref, nf_ref, first=False)


def _node_body(x_ref, agg_ref, nfin_ref, Wemb_ref, BM_ref, S_ref, Wp_ref,
               T_ref, BO_ref, BS_ref, ve_ref, nf_ref, first):
    x = x_ref[...]
    agg = agg_ref[...]

    def mm(a, b):
        return jax.lax.dot_general(a, b, (((1,), (0,)), ((), ())),
                                   preferred_element_type=jnp.float32)

    if first:
        h = mm(x, Wemb_ref[...])
        nf_cur = jnp.concatenate(
            [h, jnp.zeros((NBN, C * SH - C), jnp.float32)], axis=1)
    else:
        nf_cur = nfin_ref[...]

    bf = jnp.bfloat16
    mid = mm(agg.astype(bf), BM_ref[...])
    inv = mm(mid * mid, S_ref[...])
    g = jax.nn.silu(mm(inv, Wp_ref[...]))
    gt = mm(g, T_ref[...])
    ew = mm(x, ve_ref[...])
    sc = mm(nf_cur.astype(bf), BS_ref[...]) * ew
    nf_ref[...] = mm(mid.astype(bf), BO_ref[...]) * gt + sc


def _node_call(x, agg, nfin, Wemb, BM, S, Wp, T, BO, BS, ve, first):
    full = lambda shape: pl.BlockSpec(shape, lambda b: (0, 0))
    in_specs = [
        pl.BlockSpec((NBN, 16), lambda b: (b, 0)),
        pl.BlockSpec((NBN, C * SH), lambda b: (b, 0)),
    ]
    args = [x, agg]
    if first:
        kern = _node_kernel_first
        in_specs.append(full((16, C)))
        args.append(Wemb)
    else:
        kern = _node_kernel_rest
        in_specs.append(pl.BlockSpec((NBN, C * SH), lambda b: (b, 0)))
        args.append(nfin)
    in_specs += [full((C * SH, C * SH)), full((C * SH, C * 4)),
                 full((C * 4, C)), full((C, C * SH)),
                 full((C * SH, C * SH)), full((C * SH, C * SH)),
                 full((16, 1))]
    args += [BM, S, Wp, T, BO, BS, ve]
    return pl.pallas_call(
        kern,
        grid=(NBLK_N,),
        in_specs=in_specs,
        out_specs=pl.BlockSpec((NBN, C * SH), lambda b: (b, 0)),
        out_shape=jax.ShapeDtypeStruct((NPAD, C * SH), jnp.float32),
        compiler_params=pltpu.CompilerParams(
            dimension_semantics=("arbitrary",)),
        interpret=_INTERPRET,
    )(*args)


def _pool_kernel(nf_ref, bat_ref, PERM_ref, out_ref, acc, cnt):
    b = pl.program_id(0)

    @pl.when(b == 0)
    def _():
        acc[...] = jnp.zeros_like(acc)
        cnt[...] = jnp.zeros_like(cnt)

    bat = bat_ref[0]
    iot = jax.lax.broadcasted_iota(jnp.int32, (NG, NBN), 0)
    oh = (jnp.broadcast_to(bat, (NG, NBN)) == iot).astype(jnp.bfloat16)
    nfb = nf_ref[...].astype(jnp.bfloat16)

    def mm(a, b_, pt=jnp.float32):
        return jax.lax.dot_general(a, b_, (((1,), (0,)), ((), ())),
                                   preferred_element_type=pt)

    acc[...] = acc[...] + mm(oh, nfb)
    ones = jnp.ones((NBN, 128), jnp.bfloat16)
    cnt[...] = cnt[...] + mm(oh, ones)

    @pl.when(b == NBLK_N - 1)
    def _():
        c = jnp.maximum(cnt[:, 0:1], 1.0)
        out_ref[...] = mm(acc[...] / c, PERM_ref[...])


def _pool_call(nf, bat3, PERM):
    return pl.pallas_call(
        _pool_kernel,
        grid=(NBLK_N,),
        in_specs=[
            pl.BlockSpec((NBN, C * SH), lambda b: (b, 0)),
            pl.BlockSpec((1, 1, NBN), lambda b: (b, 0, 0)),
            pl.BlockSpec((C * SH, C * SH), lambda b: (0, 0)),
        ],
        out_specs=pl.BlockSpec((NG, C * SH), lambda b: (0, 0)),
        out_shape=jax.ShapeDtypeStruct((NG, C * SH), jnp.float32),
        scratch_shapes=[pltpu.VMEM((NG, C * SH), jnp.float32),
                        pltpu.VMEM((NG, 128), jnp.float32)],
        compiler_params=pltpu.CompilerParams(
            dimension_semantics=("arbitrary",)),
        interpret=_INTERPRET,
    )(nf, bat3, PERM)


def _sc_gather(table, idx, chunk):
    V, D = table.shape
    B = idx.shape[0]
    NW = 32
    b_per_w = B // NW
    nchunk = b_per_w // chunk
    mesh = plsc.VectorSubcoreMesh(core_axis_name="c", subcore_axis_name="s")

    @functools.partial(
        pl.kernel, mesh=mesh,
        out_type=jax.ShapeDtypeStruct((B, D), jnp.float32),
        compiler_params=pltpu.CompilerParams(use_tc_tiling_on_sc=True),
        scratch_types=[
            pltpu.VMEM((b_per_w,), jnp.int32),
            pltpu.VMEM((chunk, D), jnp.float32),
            pltpu.VMEM((chunk, D), jnp.float32),
            pltpu.SemaphoreType.DMA,
            pltpu.SemaphoreType.DMA,
        ],
    )
    def k(table_hbm, idx_hbm, out_hbm, idx_v, rows_a, rows_b, sem_a, sem_b):
        wid = lax.axis_index("s") * 2 + lax.axis_index("c")
        wbase = wid * b_per_w
        pltpu.sync_copy(idx_hbm.at[pl.ds(wbase, b_per_w)], idx_v)
        bufs = [rows_a, rows_b]
        sems = [sem_a, sem_b]
        handles = [None, None]
        for ci in range(nchunk + 1):
            if ci < nchunk:
                handles[ci % 2] = pltpu.async_copy(
                    table_hbm.at[idx_v.at[pl.ds(ci * chunk, chunk)]],
                    bufs[ci % 2], sems[ci % 2])
            if ci >= 1:
                pv = ci - 1
                handles[pv % 2].wait()
                pltpu.sync_copy(bufs[pv % 2],
                                out_hbm.at[pl.ds(wbase + pv * chunk, chunk)])

    return k(table, idx)


def _embed_kernel(x_ref, W_ref, h_ref):
    h_ref[...] = jax.lax.dot_general(
        x_ref[...], W_ref[...], (((1,), (0,)), ((), ())),
        preferred_element_type=jnp.float32)


def _embed_call(x_pad, Wemb128):
    return pl.pallas_call(
        _embed_kernel,
        out_shape=jax.ShapeDtypeStruct((NPAD, 128), jnp.float32),
        interpret=_INTERPRET,
    )(x_pad, Wemb128)


def _big_blockdiag(w4, scale=1.0):
    z = jnp.zeros((C, C), jnp.float32)
    return jnp.concatenate(
        [jnp.concatenate([z] * m + [w4[LOF[m]] * scale]
                         + [z] * (SH - 1 - m), axis=1)
         for m in range(SH)], axis=0)


def kernel(x, pos, batch, edge_index, W_embed, W1_0, W2_0, W3_0, mix_0,
           Wsc_0, velem_0, Wprod_0, Wout_0, W1_1, W2_1, W3_1, mix_1, Wsc_1,
           velem_1, Wprod_1, Wout_1):
    f32 = jnp.float32
    x = x.astype(f32)
    j = edge_index[0].astype(jnp.int32)
    i = edge_index[1].astype(jnp.int32)
    bat = batch.astype(jnp.int32)

    order = jnp.argsort(i)
    i_s = i[order]
    j_s = j[order]
    bases = ((i_s[::EB] // 8) * 8).astype(jnp.int32)
    ids3 = i_s.reshape(NBLK_E, 1, EB)

    ppos = jnp.pad(pos.astype(f32), ((0, 0), (0, 125)))
    ppj = _sc_gather(ppos, j_s, 200)
    ppi = _sc_gather(ppos, i_s, 200)
    geom = _geom_call(ppj, ppi)
    REPc = jnp.asarray(_REP_CONST)
    SELc = jnp.asarray(_SEL_CONST).astype(jnp.bfloat16)
    EXPc = jnp.asarray(_EXPAND_CONST).astype(jnp.bfloat16)

    x_pad = jnp.pad(x, ((0, NPAD - N), (0, 16 - NE)))
    Wemb_pad = jnp.pad(W_embed.astype(f32), ((0, 16 - NE), (0, 0)))

    S = jnp.asarray(_S_CONST)
    T = jnp.asarray(_TILE_CONST)
    PERM = jnp.asarray(_PERM_CONST)

    Wemb128 = jnp.pad(Wemb_pad, ((0, 0), (0, 128 - C)))
    h = _embed_call(x_pad, Wemb128)
    bat3 = jnp.pad(bat, (0, NPAD - N), constant_values=NG).reshape(
        NBLK_N, 1, NBN)

    layer_params = [
        (W1_0, W2_0, W3_0, mix_0, Wsc_0, velem_0, Wprod_0, Wout_0),
        (W1_1, W2_1, W3_1, mix_1, Wsc_1, velem_1, Wprod_1, Wout_1),
    ]

    nf = None
    s_table = h[:N]
    for li, (W1, W2, W3, mix, Wsc, velem, Wprod, Wout) in enumerate(
            layer_params):
        W3p = W3.astype(f32)[:, _W3PERM].astype(jnp.bfloat16)
        BM = _big_blockdiag(mix.astype(f32),
                            scale=1.0 / AVG_NEIGH).astype(jnp.bfloat16)
        BS = _big_blockdiag(Wsc.astype(f32)).astype(jnp.bfloat16)
        BO = _big_blockdiag(Wout.astype(f32)).astype(jnp.bfloat16)
        ve = jnp.pad(velem.astype(f32), (0, 16 - NE)).reshape(16, 1)

        sj = _sc_gather(s_table, j_s, 200)
        agg = _edge_call(bases, geom, ids3, sj, W1.astype(f32),
                         W2.astype(f32), W3p, REPc, SELc, EXPc)
        nf = _node_call(x_pad, agg, nf, Wemb_pad, BM, S,
                        Wprod.astype(f32), T, BO, BS, ve, first=(li == 0))
        s_table = jnp.pad(nf[:N, :C], ((0, 0), (0, 128 - C)))

    return _pool_call(nf, bat3, PERM)
